# Initial kernel scaffold; baseline (speedup 1.0000x reference)
import functools

import jax
import jax.numpy as jnp
from jax import lax
from jax.experimental import pallas as pl
from jax.experimental.pallas import tpu as pltpu
from jax.experimental.pallas import tpu_sc as plsc

N = 10000
E = 320000
DIN = 128
MID = 64
FOUT = 64
NEG = 0.25
HP = 16
RB = 1000
B = 80
NWORK = 32
EPW = E // NWORK
NCHUNK = EPW // B

_F32 = jnp.float32


def _tc1(x, w1, as1, ad1, w2, as2, ad2):
    def body(x_ref, w1_ref, as1_ref, ad1_ref, w2_ref, as2_ref, ad2_ref,
             h1_ref, s1_ref, d1_ref, h2_ref, s2_ref, d2_ref):
        xb = x_ref[...]
        h1 = jnp.dot(xb, w1_ref[...], preferred_element_type=_F32)
        h1_ref[...] = h1
        s1_ref[...] = jnp.dot(h1, as1_ref[...], preferred_element_type=_F32)
        d1_ref[...] = jnp.dot(h1, ad1_ref[...], preferred_element_type=_F32)
        h2 = jnp.dot(xb, w2_ref[...], preferred_element_type=_F32)
        h2_ref[...] = h2
        s2_ref[...] = jnp.dot(h2, as2_ref[...], preferred_element_type=_F32)
        d2_ref[...] = jnp.dot(h2, ad2_ref[...], preferred_element_type=_F32)

    nb = N // RB
    whole = lambda shape: pl.BlockSpec(shape, lambda i: (0,) * len(shape))
    rows = lambda w: pl.BlockSpec((RB, w), lambda i: (i, 0))
    return pl.pallas_call(
        body,
        grid=(nb,),
        in_specs=[rows(DIN), whole((DIN, MID)), whole((MID, HP)), whole((MID, HP)),
                  whole((DIN, MID)), whole((MID, HP)), whole((MID, HP))],
        out_specs=[rows(MID), rows(HP), rows(HP), rows(MID), rows(HP), rows(HP)],
        out_shape=[jax.ShapeDtypeStruct((N, MID), _F32),
                   jax.ShapeDtypeStruct((N, HP), _F32),
                   jax.ShapeDtypeStruct((N, HP), _F32),
                   jax.ShapeDtypeStruct((N, MID), _F32),
                   jax.ShapeDtypeStruct((N, HP), _F32),
                   jax.ShapeDtypeStruct((N, HP), _F32)],
    )(x, w1, as1, ad1, w2, as2, ad2)


def _edge_pair(nhead):
    wide = nhead == 8
    den_shape = (N, HP) if wide else (N,)
    row_shape = (B, HP) if wide else (B,)
    pd_shape = (2, N, HP) if wide else (2, N)

    mesh = plsc.VectorSubcoreMesh(core_axis_name="c", subcore_axis_name="s")
    out_type = (
        jax.ShapeDtypeStruct((2, N, MID), _F32),
        jax.ShapeDtypeStruct(pd_shape, _F32),
        jax.ShapeDtypeStruct((2, N, MID), _F32),
        jax.ShapeDtypeStruct(pd_shape, _F32),
    )
    scratch = [
        pltpu.VMEM((B,), jnp.int32),
        pltpu.VMEM((B,), jnp.int32),
        pltpu.VMEM(row_shape, _F32),
        pltpu.VMEM(row_shape, _F32),
        pltpu.VMEM(row_shape, _F32),
        pltpu.VMEM((B, MID), _F32),
        pltpu.VMEM((B, MID), _F32),
        pltpu.VMEM_SHARED((N, MID), _F32),
        pltpu.VMEM_SHARED(den_shape, _F32),
        pltpu.SemaphoreType.DMA,
        pltpu.SemaphoreType.DMA,
        pltpu.SemaphoreType.DMA,
    ]

    def body(srcA, dstA, sA, dA, hA, srcB, dstB, sB, dB, hB, zo, zd,
             poA, pdA, poB, pdB,
             idx_s, idx_d, srow, drow, ex, hrow, msg, acc_out, acc_den,
             sem0, sem1, sem2):
        cid = lax.axis_index("c")
        sid = lax.axis_index("s")
        wid = sid * 2 + cid
        ebase = wid * EPW
        iota = lax.iota(jnp.int32, 16)
        colpat = [lax.shift_right_logical(iota, 3) + 2 * v for v in range(4)]

        for (srcR, dstR, sR, dR, hR, poR, pdR) in (
                (srcA, dstA, sA, dA, hA, poA, pdA),
                (srcB, dstB, sB, dB, hB, poB, pdB)):

            @pl.when(sid == 0)
            def _():
                pltpu.sync_copy(zo, acc_out)
                pltpu.sync_copy(zd, acc_den)
            plsc.subcore_barrier()

            def chunk(i, carry):
                base = ebase + i * B
                pltpu.sync_copy(srcR.at[pl.ds(base, B)], idx_s)
                pltpu.sync_copy(dstR.at[pl.ds(base, B)], idx_d)
                c1 = pltpu.async_copy(sR.at[idx_s], srow, sem0)
                c2 = pltpu.async_copy(dR.at[idx_d], drow, sem1)
                c3 = pltpu.async_copy(hR.at[idx_s], hrow, sem2)
                c1.wait()
                c2.wait()

                if wide:
                    def erow(b, c):
                        ev = srow[b] + drow[b]
                        ev = jnp.where(ev >= 0.0, ev, NEG * ev)
                        ex[b] = jnp.exp(ev)
                        return c
                    lax.fori_loop(0, B, erow, 0)
                else:
                    def erow(k, c):
                        sl = pl.ds(k * 16, 16)
                        ev = srow[sl] + drow[sl]
                        ev = jnp.where(ev >= 0.0, ev, NEG * ev)
                        ex[sl] = jnp.exp(ev)
                        return c
                    lax.fori_loop(0, B // 16, erow, 0)

                pltpu.sync_copy(ex, acc_den.at[idx_d], add=True)
                c3.wait()

                if wide:
                    def mrow(b, c):
                        bvec = jnp.full((16,), b, jnp.int32)
                        for v in range(4):
                            hv = hrow[b, pl.ds(v * 16, 16)]
                            exv = plsc.load_gather(ex, [bvec, colpat[v]])
                            msg[b, pl.ds(v * 16, 16)] = hv * exv
                        return c
                    lax.fori_loop(0, B, mrow, 0)
                else:
                    def mrow(b, c):
                        exs = ex[b]
                        for v in range(4):
                            sl = pl.ds(v * 16, 16)
                            msg[b, sl] = hrow[b, sl] * exs
                        return c
                    lax.fori_loop(0, B, mrow, 0)

                pltpu.sync_copy(msg, acc_out.at[idx_d], add=True)
                return carry

            lax.fori_loop(0, NCHUNK, chunk, 0)
            plsc.subcore_barrier()

            @pl.when(sid == 0)
            def _():
                pltpu.sync_copy(acc_out, poR.at[cid])
                pltpu.sync_copy(acc_den, pdR.at[cid])
            plsc.subcore_barrier()

    return pl.kernel(body, out_type=out_type, mesh=mesh, scratch_types=scratch)


_edge8 = _edge_pair(8)
_edge1 = _edge_pair(1)


def _tc2(poA, pdA, poB, pdB, b1, b2, wg1a, wg1b, expd, w12, sd12w, w22, sd22w):
    def body(poA_ref, pdA_ref, poB_ref, pdB_ref, b1_ref, b2_ref, wg1a_ref,
             wg1b_ref, expd_ref, w12_ref, sd12w_ref, w22_ref, sd22w_ref,
             h12_ref, sd12_ref, h22_ref, sd22_ref):
        expd_m = expd_ref[...]
        o1 = poA_ref[0] + poA_ref[1]
        rec1 = 1.0 / (pdA_ref[0] + pdA_ref[1] + 1e-16)
        ns11 = o1 * jnp.dot(rec1, expd_m, preferred_element_type=_F32) + b1_ref[...]
        ns11 = jnp.where(ns11 > 0.0, ns11, jnp.exp(ns11) - 1.0)
        o2 = poB_ref[0] + poB_ref[1]
        rec2 = 1.0 / (pdB_ref[0] + pdB_ref[1] + 1e-16)
        ns21 = o2 * jnp.dot(rec2, expd_m, preferred_element_type=_F32) + b2_ref[...]
        ns21 = jnp.where(ns21 > 0.0, ns21, jnp.exp(ns21) - 1.0)
        zl = (jnp.dot(ns11, wg1a_ref[...], preferred_element_type=_F32) +
              jnp.dot(ns21, wg1b_ref[...], preferred_element_type=_F32))
        z = 1.0 / (1.0 + jnp.exp(-zl))
        midv = z * ns11 + (1.0 - z) * ns21
        h12 = jnp.dot(midv, w12_ref[...], preferred_element_type=_F32)
        h12_ref[...] = h12
        sd12_ref[...] = jnp.dot(h12, sd12w_ref[...], preferred_element_type=_F32)
        h22 = jnp.dot(midv, w22_ref[...], preferred_element_type=_F32)
        h22_ref[...] = h22
        sd22_ref[...] = jnp.dot(h22, sd22w_ref[...], preferred_element_type=_F32)

    nb = N // RB
    whole = lambda shape: pl.BlockSpec(shape, lambda i: (0,) * len(shape))
    rows = lambda w: pl.BlockSpec((RB, w), lambda i: (i, 0))
    prow = lambda w: pl.BlockSpec((2, RB, w), lambda i: (0, i, 0))
    return pl.pallas_call(
        body,
        grid=(nb,),
        in_specs=[prow(MID), prow(HP), prow(MID), prow(HP),
                  whole((1, MID)), whole((1, MID)),
                  whole((MID, MID)), whole((MID, MID)), whole((HP, MID)),
                  whole((MID, MID)), whole((MID, 8)),
                  whole((MID, MID)), whole((MID, 8))],
        out_specs=[rows(MID), rows(8), rows(MID), rows(8)],
        out_shape=[jax.ShapeDtypeStruct((N, MID), _F32),
                   jax.ShapeDtypeStruct((N, 8), _F32),
                   jax.ShapeDtypeStruct((N, MID), _F32),
                   jax.ShapeDtypeStruct((N, 8), _F32)],
    )(poA, pdA, poB, pdB, b1, b2, wg1a, wg1b, expd, w12, sd12w, w22, sd22w)


def _tc3(poA, pdA, poB, pdB, b1, b2, wg2a, wg2b):
    def body(poA_ref, pdA_ref, poB_ref, pdB_ref, b1_ref, b2_ref,
             wg2a_ref, wg2b_ref, out_ref):
        o1 = poA_ref[0] + poA_ref[1]
        den1 = pdA_ref[0] + pdA_ref[1]
        ns12 = o1 * (1.0 / (den1 + 1e-16)) + b1_ref[...]
        o2 = poB_ref[0] + poB_ref[1]
        den2 = pdB_ref[0] + pdB_ref[1]
        ns22 = o2 * (1.0 / (den2 + 1e-16)) + b2_ref[...]
        zl = (jnp.dot(ns12, wg2a_ref[...], preferred_element_type=_F32) +
              jnp.dot(ns22, wg2b_ref[...], preferred_element_type=_F32))
        z = 1.0 / (1.0 + jnp.exp(-zl))
        outv = z * ns12 + (1.0 - z) * ns22
        m = jnp.max(outv, axis=1, keepdims=True)
        sh = outv - m
        lse = jnp.log(jnp.sum(jnp.exp(sh), axis=1, keepdims=True))
        out_ref[...] = sh - lse

    nb = N // RB
    whole = lambda shape: pl.BlockSpec(shape, lambda i: (0,) * len(shape))
    rows = lambda w: pl.BlockSpec((RB, w), lambda i: (i, 0))
    prow = lambda w: pl.BlockSpec((2, RB, w), lambda i: (0, i, 0))
    return pl.pallas_call(
        body,
        grid=(nb,),
        in_specs=[prow(MID), prow(1), prow(MID), prow(1),
                  whole((1, MID)), whole((1, MID)),
                  whole((MID, MID)), whole((MID, MID))],
        out_specs=rows(MID),
        out_shape=jax.ShapeDtypeStruct((N, MID), _F32),
    )(poA, pdA, poB, pdB, b1, b2, wg2a, wg2b)


def _blockdiag(a):
    h, c = a.shape
    eye = jnp.eye(h, dtype=_F32)
    m = (a[:, :, None] * eye[:, None, :]).reshape(h * c, h)
    return jnp.pad(m, ((0, 0), (0, HP - h)))


def kernel(node_feature, one_adj_list, two_adj_list, W11, a_src11, a_dst11,
           b11, W21, a_src21, a_dst21, b21, Wg1, W12, a_src12, a_dst12, b12,
           W22, a_src22, a_dst22, b22, Wg2):
    srcA, dstA = one_adj_list[0], one_adj_list[1]
    srcB, dstB = two_adj_list[0], two_adj_list[1]

    h1, s1, d1, h2, s2, d2 = _tc1(
        node_feature, W11, _blockdiag(a_src11), _blockdiag(a_dst11),
        W21, _blockdiag(a_src21), _blockdiag(a_dst21))

    zo = jnp.zeros((N, MID), _F32)
    z16 = jnp.zeros((N, HP), _F32)
    poA, pdA, poB, pdB = _edge8(srcA, dstA, s1, d1, h1,
                                srcB, dstB, s2, d2, h2, zo, z16)

    expd = jnp.pad(jnp.repeat(jnp.eye(8, dtype=_F32), 8, axis=1),
                   ((0, HP - 8), (0, 0)))
    sd12w = jnp.pad(jnp.concatenate(
        [a_src12.reshape(MID, 1), a_dst12.reshape(MID, 1)], axis=1),
        ((0, 0), (0, 6)))
    sd22w = jnp.pad(jnp.concatenate(
        [a_src22.reshape(MID, 1), a_dst22.reshape(MID, 1)], axis=1),
        ((0, 0), (0, 6)))
    h12, sd12, h22, sd22 = _tc2(
        poA, pdA, poB, pdB, b11.reshape(1, MID), b21.reshape(1, MID),
        Wg1[:MID], Wg1[MID:], expd, W12, sd12w, W22, sd22w)

    zN = jnp.zeros((N,), _F32)
    poA2, pdA2, poB2, pdB2 = _edge1(
        srcA, dstA, sd12[:, 0], sd12[:, 1], h12,
        srcB, dstB, sd22[:, 0], sd22[:, 1], h22, zo, zN)

    return _tc3(poA2, pdA2.reshape(2, N, 1), poB2, pdB2.reshape(2, N, 1),
                b12.reshape(1, MID), b22.reshape(1, MID), Wg2[:MID], Wg2[MID:])


# speedup vs baseline: 46.8898x; 46.8898x over previous
import functools

import jax
import jax.numpy as jnp
from jax import lax
from jax.experimental import pallas as pl
from jax.experimental.pallas import tpu as pltpu
from jax.experimental.pallas import tpu_sc as plsc

N = 10000
E = 320000
DIN = 128
MID = 64
FOUT = 64
NEG = 0.25
HP = 16
RB = 1000
B = 80
NWORK = 32
EPW = E // NWORK
NCHUNK = EPW // B

_F32 = jnp.float32


def _tc1(x, w1, as1, ad1, w2, as2, ad2):
    def body(x_ref, w1_ref, as1_ref, ad1_ref, w2_ref, as2_ref, ad2_ref,
             h1_ref, s1_ref, d1_ref, h2_ref, s2_ref, d2_ref):
        xb = x_ref[...]
        h1 = jnp.dot(xb, w1_ref[...], preferred_element_type=_F32)
        h1_ref[...] = h1
        s1_ref[...] = jnp.dot(h1, as1_ref[...], preferred_element_type=_F32)
        d1_ref[...] = jnp.dot(h1, ad1_ref[...], preferred_element_type=_F32)
        h2 = jnp.dot(xb, w2_ref[...], preferred_element_type=_F32)
        h2_ref[...] = h2
        s2_ref[...] = jnp.dot(h2, as2_ref[...], preferred_element_type=_F32)
        d2_ref[...] = jnp.dot(h2, ad2_ref[...], preferred_element_type=_F32)

    nb = N // RB
    whole = lambda shape: pl.BlockSpec(shape, lambda i: (0,) * len(shape))
    rows = lambda w: pl.BlockSpec((RB, w), lambda i: (i, 0))
    return pl.pallas_call(
        body,
        grid=(nb,),
        in_specs=[rows(DIN), whole((DIN, MID)), whole((MID, HP)), whole((MID, HP)),
                  whole((DIN, MID)), whole((MID, HP)), whole((MID, HP))],
        out_specs=[rows(MID), rows(HP), rows(HP), rows(MID), rows(HP), rows(HP)],
        out_shape=[jax.ShapeDtypeStruct((N, MID), _F32),
                   jax.ShapeDtypeStruct((N, HP), _F32),
                   jax.ShapeDtypeStruct((N, HP), _F32),
                   jax.ShapeDtypeStruct((N, MID), _F32),
                   jax.ShapeDtypeStruct((N, HP), _F32),
                   jax.ShapeDtypeStruct((N, HP), _F32)],
    )(x, w1, as1, ad1, w2, as2, ad2)


def _edge_pair(nhead):
    wide = nhead == 8
    den_shape = (N, HP) if wide else (N,)
    row_shape = (B, HP) if wide else (B,)
    pd_shape = (2, N, HP) if wide else (2, N)

    mesh = plsc.VectorSubcoreMesh(core_axis_name="c", subcore_axis_name="s")
    out_type = (
        jax.ShapeDtypeStruct((2, N, MID), _F32),
        jax.ShapeDtypeStruct(pd_shape, _F32),
        jax.ShapeDtypeStruct((2, N, MID), _F32),
        jax.ShapeDtypeStruct(pd_shape, _F32),
    )
    scratch = [
        pltpu.VMEM((B,), jnp.int32),
        pltpu.VMEM((B,), jnp.int32),
        pltpu.VMEM(row_shape, _F32),
        pltpu.VMEM(row_shape, _F32),
        pltpu.VMEM(row_shape, _F32),
        pltpu.VMEM((B, MID), _F32),
        pltpu.VMEM((B, MID), _F32),
        pltpu.VMEM_SHARED((N, MID), _F32),
        pltpu.VMEM_SHARED(den_shape, _F32),
        pltpu.SemaphoreType.DMA,
        pltpu.SemaphoreType.DMA,
        pltpu.SemaphoreType.DMA,
    ]

    def body(srcA, dstA, sA, dA, hA, srcB, dstB, sB, dB, hB, zo, zd,
             poA, pdA, poB, pdB,
             idx_s, idx_d, srow, drow, ex, hrow, msg, acc_out, acc_den,
             sem0, sem1, sem2):
        cid = lax.axis_index("c")
        sid = lax.axis_index("s")
        wid = sid * 2 + cid
        ebase = wid * EPW
        iota = lax.iota(jnp.int32, 16)
        colpat = [lax.shift_right_logical(iota, 3) + 2 * v for v in range(4)]

        for (srcR, dstR, sR, dR, hR, poR, pdR) in (
                (srcA, dstA, sA, dA, hA, poA, pdA),
                (srcB, dstB, sB, dB, hB, poB, pdB)):

            @pl.when(sid == 0)
            def _():
                pltpu.sync_copy(zo, acc_out)
                pltpu.sync_copy(zd, acc_den)
            plsc.subcore_barrier()

            def chunk(i, carry):
                base = ebase + i * B
                pltpu.sync_copy(srcR.at[pl.ds(base, B)], idx_s)
                pltpu.sync_copy(dstR.at[pl.ds(base, B)], idx_d)
                c1 = pltpu.async_copy(sR.at[idx_s], srow, sem0)
                c2 = pltpu.async_copy(dR.at[idx_d], drow, sem1)
                c3 = pltpu.async_copy(hR.at[idx_s], hrow, sem2)
                c1.wait()
                c2.wait()

                if wide:
                    def erow(b, c):
                        ev = srow[b] + drow[b]
                        ev = jnp.where(ev >= 0.0, ev, NEG * ev)
                        ex[b] = jnp.exp(ev)
                        return c
                    lax.fori_loop(0, B, erow, 0)
                else:
                    def erow(k, c):
                        sl = pl.ds(k * 16, 16)
                        ev = srow[sl] + drow[sl]
                        ev = jnp.where(ev >= 0.0, ev, NEG * ev)
                        ex[sl] = jnp.exp(ev)
                        return c
                    lax.fori_loop(0, B // 16, erow, 0)

                pltpu.sync_copy(ex, acc_den.at[idx_d], add=True)
                c3.wait()

                if wide:
                    def mrow(b, c):
                        bvec = jnp.full((16,), b, jnp.int32)
                        for v in range(4):
                            hv = hrow[b, pl.ds(v * 16, 16)]
                            exv = plsc.load_gather(ex, [bvec, colpat[v]])
                            msg[b, pl.ds(v * 16, 16)] = hv * exv
                        return c
                    lax.fori_loop(0, B, mrow, 0)
                else:
                    def mrow(b, c):
                        exv = plsc.load_gather(ex, [jnp.full((16,), b, jnp.int32)])
                        for v in range(4):
                            sl = pl.ds(v * 16, 16)
                            msg[b, sl] = hrow[b, sl] * exv
                        return c
                    lax.fori_loop(0, B, mrow, 0)

                pltpu.sync_copy(msg, acc_out.at[idx_d], add=True)
                return carry

            lax.fori_loop(0, NCHUNK, chunk, 0)
            plsc.subcore_barrier()

            @pl.when(sid == 0)
            def _():
                pltpu.sync_copy(acc_out, poR.at[cid])
                pltpu.sync_copy(acc_den, pdR.at[cid])
            plsc.subcore_barrier()

    return pl.kernel(
        body, out_type=out_type, mesh=mesh, scratch_types=scratch,
        compiler_params=pltpu.CompilerParams(
            needs_layout_passes=False, use_tc_tiling_on_sc=False))


_edge8 = _edge_pair(8)
_edge1 = _edge_pair(1)


def _tc2(poA, pdA, poB, pdB, b1, b2, wg1a, wg1b, expd, w12, sd12w, w22, sd22w):
    def body(poA_ref, pdA_ref, poB_ref, pdB_ref, b1_ref, b2_ref, wg1a_ref,
             wg1b_ref, expd_ref, w12_ref, sd12w_ref, w22_ref, sd22w_ref,
             h12_ref, sd12_ref, h22_ref, sd22_ref):
        expd_m = expd_ref[...]
        o1 = poA_ref[0] + poA_ref[1]
        rec1 = 1.0 / (pdA_ref[0] + pdA_ref[1] + 1e-16)
        ns11 = o1 * jnp.dot(rec1, expd_m, preferred_element_type=_F32) + b1_ref[...]
        ns11 = jnp.where(ns11 > 0.0, ns11, jnp.exp(ns11) - 1.0)
        o2 = poB_ref[0] + poB_ref[1]
        rec2 = 1.0 / (pdB_ref[0] + pdB_ref[1] + 1e-16)
        ns21 = o2 * jnp.dot(rec2, expd_m, preferred_element_type=_F32) + b2_ref[...]
        ns21 = jnp.where(ns21 > 0.0, ns21, jnp.exp(ns21) - 1.0)
        zl = (jnp.dot(ns11, wg1a_ref[...], preferred_element_type=_F32) +
              jnp.dot(ns21, wg1b_ref[...], preferred_element_type=_F32))
        z = 1.0 / (1.0 + jnp.exp(-zl))
        midv = z * ns11 + (1.0 - z) * ns21
        h12 = jnp.dot(midv, w12_ref[...], preferred_element_type=_F32)
        h12_ref[...] = h12
        sd12_ref[...] = jnp.dot(h12, sd12w_ref[...], preferred_element_type=_F32)
        h22 = jnp.dot(midv, w22_ref[...], preferred_element_type=_F32)
        h22_ref[...] = h22
        sd22_ref[...] = jnp.dot(h22, sd22w_ref[...], preferred_element_type=_F32)

    nb = N // RB
    whole = lambda shape: pl.BlockSpec(shape, lambda i: (0,) * len(shape))
    rows = lambda w: pl.BlockSpec((RB, w), lambda i: (i, 0))
    prow = lambda w: pl.BlockSpec((2, RB, w), lambda i: (0, i, 0))
    return pl.pallas_call(
        body,
        grid=(nb,),
        in_specs=[prow(MID), prow(HP), prow(MID), prow(HP),
                  whole((1, MID)), whole((1, MID)),
                  whole((MID, MID)), whole((MID, MID)), whole((HP, MID)),
                  whole((MID, MID)), whole((MID, 8)),
                  whole((MID, MID)), whole((MID, 8))],
        out_specs=[rows(MID), rows(8), rows(MID), rows(8)],
        out_shape=[jax.ShapeDtypeStruct((N, MID), _F32),
                   jax.ShapeDtypeStruct((N, 8), _F32),
                   jax.ShapeDtypeStruct((N, MID), _F32),
                   jax.ShapeDtypeStruct((N, 8), _F32)],
    )(poA, pdA, poB, pdB, b1, b2, wg1a, wg1b, expd, w12, sd12w, w22, sd22w)


def _tc3(poA, pdA, poB, pdB, b1, b2, wg2a, wg2b):
    def body(poA_ref, pdA_ref, poB_ref, pdB_ref, b1_ref, b2_ref,
             wg2a_ref, wg2b_ref, out_ref):
        o1 = poA_ref[0] + poA_ref[1]
        den1 = pdA_ref[0] + pdA_ref[1]
        ns12 = o1 * (1.0 / (den1 + 1e-16)) + b1_ref[...]
        o2 = poB_ref[0] + poB_ref[1]
        den2 = pdB_ref[0] + pdB_ref[1]
        ns22 = o2 * (1.0 / (den2 + 1e-16)) + b2_ref[...]
        zl = (jnp.dot(ns12, wg2a_ref[...], preferred_element_type=_F32) +
              jnp.dot(ns22, wg2b_ref[...], preferred_element_type=_F32))
        z = 1.0 / (1.0 + jnp.exp(-zl))
        outv = z * ns12 + (1.0 - z) * ns22
        m = jnp.max(outv, axis=1, keepdims=True)
        sh = outv - m
        lse = jnp.log(jnp.sum(jnp.exp(sh), axis=1, keepdims=True))
        out_ref[...] = sh - lse

    nb = N // RB
    whole = lambda shape: pl.BlockSpec(shape, lambda i: (0,) * len(shape))
    rows = lambda w: pl.BlockSpec((RB, w), lambda i: (i, 0))
    prow = lambda w: pl.BlockSpec((2, RB, w), lambda i: (0, i, 0))
    return pl.pallas_call(
        body,
        grid=(nb,),
        in_specs=[prow(MID), prow(1), prow(MID), prow(1),
                  whole((1, MID)), whole((1, MID)),
                  whole((MID, MID)), whole((MID, MID))],
        out_specs=rows(MID),
        out_shape=jax.ShapeDtypeStruct((N, MID), _F32),
    )(poA, pdA, poB, pdB, b1, b2, wg2a, wg2b)


def _blockdiag(a):
    h, c = a.shape
    eye = jnp.eye(h, dtype=_F32)
    m = (a[:, :, None] * eye[:, None, :]).reshape(h * c, h)
    return jnp.pad(m, ((0, 0), (0, HP - h)))


def kernel(node_feature, one_adj_list, two_adj_list, W11, a_src11, a_dst11,
           b11, W21, a_src21, a_dst21, b21, Wg1, W12, a_src12, a_dst12, b12,
           W22, a_src22, a_dst22, b22, Wg2):
    srcA, dstA = one_adj_list[0], one_adj_list[1]
    srcB, dstB = two_adj_list[0], two_adj_list[1]

    h1, s1, d1, h2, s2, d2 = _tc1(
        node_feature, W11, _blockdiag(a_src11), _blockdiag(a_dst11),
        W21, _blockdiag(a_src21), _blockdiag(a_dst21))

    zo = jnp.zeros((N, MID), _F32)
    z16 = jnp.zeros((N, HP), _F32)
    poA, pdA, poB, pdB = _edge8(srcA, dstA, s1, d1, h1,
                                srcB, dstB, s2, d2, h2, zo, z16)

    expd = jnp.pad(jnp.repeat(jnp.eye(8, dtype=_F32), 8, axis=1),
                   ((0, HP - 8), (0, 0)))
    sd12w = jnp.pad(jnp.concatenate(
        [a_src12.reshape(MID, 1), a_dst12.reshape(MID, 1)], axis=1),
        ((0, 0), (0, 6)))
    sd22w = jnp.pad(jnp.concatenate(
        [a_src22.reshape(MID, 1), a_dst22.reshape(MID, 1)], axis=1),
        ((0, 0), (0, 6)))
    h12, sd12, h22, sd22 = _tc2(
        poA, pdA, poB, pdB, b11.reshape(1, MID), b21.reshape(1, MID),
        Wg1[:MID], Wg1[MID:], expd, W12, sd12w, W22, sd22w)

    zN = jnp.zeros((N,), _F32)
    poA2, pdA2, poB2, pdB2 = _edge1(
        srcA, dstA, sd12[:, 0], sd12[:, 1], h12,
        srcB, dstB, sd22[:, 0], sd22[:, 1], h22, zo, zN)

    return _tc3(poA2, pdA2.reshape(2, N, 1), poB2, pdB2.reshape(2, N, 1),
                b12.reshape(1, MID), b22.reshape(1, MID), Wg2[:MID], Wg2[MID:])


# speedup vs baseline: 73.9076x; 1.5762x over previous
import functools

import jax
import jax.numpy as jnp
from jax import lax
from jax.experimental import pallas as pl
from jax.experimental.pallas import tpu as pltpu
from jax.experimental.pallas import tpu_sc as plsc

N = 10000
E = 320000
DIN = 128
MID = 64
FOUT = 64
NEG = 0.25
HP = 16
RB = 1000
B = 80
NWORK = 32
EPW = E // NWORK
NCHUNK = EPW // B

_F32 = jnp.float32


def _tc1(x, w1, as1, ad1, w2, as2, ad2):
    def body(x_ref, w1_ref, as1_ref, ad1_ref, w2_ref, as2_ref, ad2_ref,
             h1_ref, s1_ref, d1_ref, h2_ref, s2_ref, d2_ref):
        xb = x_ref[...]
        h1 = jnp.dot(xb, w1_ref[...], preferred_element_type=_F32)
        h1_ref[...] = h1
        s1_ref[...] = jnp.dot(h1, as1_ref[...], preferred_element_type=_F32)
        d1_ref[...] = jnp.dot(h1, ad1_ref[...], preferred_element_type=_F32)
        h2 = jnp.dot(xb, w2_ref[...], preferred_element_type=_F32)
        h2_ref[...] = h2
        s2_ref[...] = jnp.dot(h2, as2_ref[...], preferred_element_type=_F32)
        d2_ref[...] = jnp.dot(h2, ad2_ref[...], preferred_element_type=_F32)

    nb = N // RB
    whole = lambda shape: pl.BlockSpec(shape, lambda i: (0,) * len(shape))
    rows = lambda w: pl.BlockSpec((RB, w), lambda i: (i, 0))
    return pl.pallas_call(
        body,
        grid=(nb,),
        in_specs=[rows(DIN), whole((DIN, MID)), whole((MID, HP)), whole((MID, HP)),
                  whole((DIN, MID)), whole((MID, HP)), whole((MID, HP))],
        out_specs=[rows(MID), rows(HP), rows(HP), rows(MID), rows(HP), rows(HP)],
        out_shape=[jax.ShapeDtypeStruct((N, MID), _F32),
                   jax.ShapeDtypeStruct((N, HP), _F32),
                   jax.ShapeDtypeStruct((N, HP), _F32),
                   jax.ShapeDtypeStruct((N, MID), _F32),
                   jax.ShapeDtypeStruct((N, HP), _F32),
                   jax.ShapeDtypeStruct((N, HP), _F32)],
    )(x, w1, as1, ad1, w2, as2, ad2)


def _edge_pair(nhead):
    wide = nhead == 8
    den_shape = (N, HP) if wide else (N,)
    row_shape = (B, HP) if wide else (B,)
    pd_shape = (2, N, HP) if wide else (2, N)

    mesh = plsc.VectorSubcoreMesh(core_axis_name="c", subcore_axis_name="s")
    out_type = (
        jax.ShapeDtypeStruct((2, N, MID), _F32),
        jax.ShapeDtypeStruct(pd_shape, _F32),
        jax.ShapeDtypeStruct((2, N, MID), _F32),
        jax.ShapeDtypeStruct(pd_shape, _F32),
    )
    buf = lambda: [pltpu.VMEM(row_shape, _F32),
                   pltpu.VMEM(row_shape, _F32),
                   pltpu.VMEM(row_shape, _F32),
                   pltpu.VMEM((B, MID), _F32),
                   pltpu.VMEM((B, MID), _F32),
                   pltpu.SemaphoreType.DMA]
    scratch = [
        pltpu.VMEM((NCHUNK, B), jnp.int32),
        pltpu.VMEM((NCHUNK, B), jnp.int32),
        *buf(), *buf(),
        pltpu.VMEM_SHARED((N, MID), _F32),
        pltpu.VMEM_SHARED(den_shape, _F32),
    ]

    def body(srcA, dstA, sA, dA, hA, srcB, dstB, sB, dB, hB, zo, zd,
             poA, pdA, poB, pdB,
             ixs, ixd, sr0, dr0, ex0, hr0, ms0, g0, sr1, dr1, ex1, hr1, ms1,
             g1, acc_out, acc_den):
        cid = lax.axis_index("c")
        sid = lax.axis_index("s")
        wid = sid * 2 + cid
        rbase = wid * NCHUNK
        iota = lax.iota(jnp.int32, 16)
        colpat = [lax.shift_right_logical(iota, 3) + 2 * v for v in range(4)]
        bufs = ((sr0, dr0, ex0, hr0, ms0, g0), (sr1, dr1, ex1, hr1, ms1, g1))

        for (srcR, dstR, sR, dR, hR, poR, pdR) in (
                (srcA, dstA, sA, dA, hA, poA, pdA),
                (srcB, dstB, sB, dB, hB, poB, pdB)):

            @pl.when(sid == 0)
            def _():
                pltpu.sync_copy(zo, acc_out)
                pltpu.sync_copy(zd, acc_den)
            pltpu.sync_copy(srcR.at[pl.ds(rbase, NCHUNK)], ixs)
            pltpu.sync_copy(dstR.at[pl.ds(rbase, NCHUNK)], ixd)
            plsc.subcore_barrier()

            def fire(i, bf):
                sr, dr, exb, hr, ms, sem = bf
                pltpu.async_copy(sR.at[ixs.at[i]], sr, sem)
                pltpu.async_copy(dR.at[ixd.at[i]], dr, sem)
                pltpu.async_copy(hR.at[ixs.at[i]], hr, sem)

            def waitg(i, bf):
                sr, dr, exb, hr, ms, sem = bf
                pltpu.make_async_copy(sR.at[ixs.at[i]], sr, sem).wait()
                pltpu.make_async_copy(dR.at[ixd.at[i]], dr, sem).wait()
                pltpu.make_async_copy(hR.at[ixs.at[i]], hr, sem).wait()

            def compute(i, bf):
                sr, dr, exb, hr, ms, sem = bf
                if wide:
                    def erow(b, c):
                        ev = sr[b] + dr[b]
                        ev = jnp.where(ev >= 0.0, ev, NEG * ev)
                        exb[b] = jnp.exp(ev)
                        return c
                    lax.fori_loop(0, B, erow, 0)
                else:
                    def erow(k, c):
                        sl = pl.ds(k * 16, 16)
                        ev = sr[sl] + dr[sl]
                        ev = jnp.where(ev >= 0.0, ev, NEG * ev)
                        exb[sl] = jnp.exp(ev)
                        return c
                    lax.fori_loop(0, B // 16, erow, 0)

                pltpu.sync_copy(exb, acc_den.at[ixd.at[i]], add=True)

                if wide:
                    def mrow(b, c):
                        bvec = jnp.full((16,), b, jnp.int32)
                        for v in range(4):
                            hv = hr[b, pl.ds(v * 16, 16)]
                            exv = plsc.load_gather(exb, [bvec, colpat[v]])
                            ms[b, pl.ds(v * 16, 16)] = hv * exv
                        return c
                    lax.fori_loop(0, B, mrow, 0)
                else:
                    def mrow(b, c):
                        exv = plsc.load_gather(
                            exb, [jnp.full((16,), b, jnp.int32)])
                        for v in range(4):
                            sl = pl.ds(v * 16, 16)
                            ms[b, sl] = hr[b, sl] * exv
                        return c
                    lax.fori_loop(0, B, mrow, 0)

                pltpu.sync_copy(ms, acc_out.at[ixd.at[i]], add=True)

            fire(0, bufs[0])

            def step(k, c):
                ia = 2 * k
                fire(ia + 1, bufs[1])
                waitg(ia, bufs[0])
                compute(ia, bufs[0])
                fire(ia + 2, bufs[0])
                waitg(ia + 1, bufs[1])
                compute(ia + 1, bufs[1])
                return c

            lax.fori_loop(0, (NCHUNK - 1) // 2, step, 0)
            waitg(NCHUNK - 1, bufs[0])
            compute(NCHUNK - 1, bufs[0])
            plsc.subcore_barrier()

            @pl.when(sid == 0)
            def _():
                pltpu.sync_copy(acc_out, poR.at[cid])
                pltpu.sync_copy(acc_den, pdR.at[cid])
            plsc.subcore_barrier()

    return pl.kernel(
        body, out_type=out_type, mesh=mesh, scratch_types=scratch,
        compiler_params=pltpu.CompilerParams(
            needs_layout_passes=False, use_tc_tiling_on_sc=False))


_edge8 = _edge_pair(8)
_edge1 = _edge_pair(1)


def _tc2(poA, pdA, poB, pdB, b1, b2, wg1a, wg1b, expd, w12, sd12w, w22, sd22w):
    def body(poA_ref, pdA_ref, poB_ref, pdB_ref, b1_ref, b2_ref, wg1a_ref,
             wg1b_ref, expd_ref, w12_ref, sd12w_ref, w22_ref, sd22w_ref,
             h12_ref, sd12_ref, h22_ref, sd22_ref):
        expd_m = expd_ref[...]
        o1 = poA_ref[0] + poA_ref[1]
        rec1 = 1.0 / (pdA_ref[0] + pdA_ref[1] + 1e-16)
        ns11 = o1 * jnp.dot(rec1, expd_m, preferred_element_type=_F32) + b1_ref[...]
        ns11 = jnp.where(ns11 > 0.0, ns11, jnp.exp(ns11) - 1.0)
        o2 = poB_ref[0] + poB_ref[1]
        rec2 = 1.0 / (pdB_ref[0] + pdB_ref[1] + 1e-16)
        ns21 = o2 * jnp.dot(rec2, expd_m, preferred_element_type=_F32) + b2_ref[...]
        ns21 = jnp.where(ns21 > 0.0, ns21, jnp.exp(ns21) - 1.0)
        zl = (jnp.dot(ns11, wg1a_ref[...], preferred_element_type=_F32) +
              jnp.dot(ns21, wg1b_ref[...], preferred_element_type=_F32))
        z = 1.0 / (1.0 + jnp.exp(-zl))
        midv = z * ns11 + (1.0 - z) * ns21
        h12 = jnp.dot(midv, w12_ref[...], preferred_element_type=_F32)
        h12_ref[...] = h12
        sd12_ref[...] = jnp.dot(h12, sd12w_ref[...], preferred_element_type=_F32)
        h22 = jnp.dot(midv, w22_ref[...], preferred_element_type=_F32)
        h22_ref[...] = h22
        sd22_ref[...] = jnp.dot(h22, sd22w_ref[...], preferred_element_type=_F32)

    nb = N // RB
    whole = lambda shape: pl.BlockSpec(shape, lambda i: (0,) * len(shape))
    rows = lambda w: pl.BlockSpec((RB, w), lambda i: (i, 0))
    prow = lambda w: pl.BlockSpec((2, RB, w), lambda i: (0, i, 0))
    return pl.pallas_call(
        body,
        grid=(nb,),
        in_specs=[prow(MID), prow(HP), prow(MID), prow(HP),
                  whole((1, MID)), whole((1, MID)),
                  whole((MID, MID)), whole((MID, MID)), whole((HP, MID)),
                  whole((MID, MID)), whole((MID, 8)),
                  whole((MID, MID)), whole((MID, 8))],
        out_specs=[rows(MID), rows(8), rows(MID), rows(8)],
        out_shape=[jax.ShapeDtypeStruct((N, MID), _F32),
                   jax.ShapeDtypeStruct((N, 8), _F32),
                   jax.ShapeDtypeStruct((N, MID), _F32),
                   jax.ShapeDtypeStruct((N, 8), _F32)],
    )(poA, pdA, poB, pdB, b1, b2, wg1a, wg1b, expd, w12, sd12w, w22, sd22w)


def _tc3(poA, pdA, poB, pdB, b1, b2, wg2a, wg2b):
    def body(poA_ref, pdA_ref, poB_ref, pdB_ref, b1_ref, b2_ref,
             wg2a_ref, wg2b_ref, out_ref):
        o1 = poA_ref[0] + poA_ref[1]
        den1 = pdA_ref[0] + pdA_ref[1]
        ns12 = o1 * (1.0 / (den1 + 1e-16)) + b1_ref[...]
        o2 = poB_ref[0] + poB_ref[1]
        den2 = pdB_ref[0] + pdB_ref[1]
        ns22 = o2 * (1.0 / (den2 + 1e-16)) + b2_ref[...]
        zl = (jnp.dot(ns12, wg2a_ref[...], preferred_element_type=_F32) +
              jnp.dot(ns22, wg2b_ref[...], preferred_element_type=_F32))
        z = 1.0 / (1.0 + jnp.exp(-zl))
        outv = z * ns12 + (1.0 - z) * ns22
        m = jnp.max(outv, axis=1, keepdims=True)
        sh = outv - m
        lse = jnp.log(jnp.sum(jnp.exp(sh), axis=1, keepdims=True))
        out_ref[...] = sh - lse

    nb = N // RB
    whole = lambda shape: pl.BlockSpec(shape, lambda i: (0,) * len(shape))
    rows = lambda w: pl.BlockSpec((RB, w), lambda i: (i, 0))
    prow = lambda w: pl.BlockSpec((2, RB, w), lambda i: (0, i, 0))
    return pl.pallas_call(
        body,
        grid=(nb,),
        in_specs=[prow(MID), prow(1), prow(MID), prow(1),
                  whole((1, MID)), whole((1, MID)),
                  whole((MID, MID)), whole((MID, MID))],
        out_specs=rows(MID),
        out_shape=jax.ShapeDtypeStruct((N, MID), _F32),
    )(poA, pdA, poB, pdB, b1, b2, wg2a, wg2b)


def _blockdiag(a):
    h, c = a.shape
    eye = jnp.eye(h, dtype=_F32)
    m = (a[:, :, None] * eye[:, None, :]).reshape(h * c, h)
    return jnp.pad(m, ((0, 0), (0, HP - h)))


def kernel(node_feature, one_adj_list, two_adj_list, W11, a_src11, a_dst11,
           b11, W21, a_src21, a_dst21, b21, Wg1, W12, a_src12, a_dst12, b12,
           W22, a_src22, a_dst22, b22, Wg2):
    srcA = one_adj_list[0].reshape(E // B, B)
    dstA = one_adj_list[1].reshape(E // B, B)
    srcB = two_adj_list[0].reshape(E // B, B)
    dstB = two_adj_list[1].reshape(E // B, B)

    h1, s1, d1, h2, s2, d2 = _tc1(
        node_feature, W11, _blockdiag(a_src11), _blockdiag(a_dst11),
        W21, _blockdiag(a_src21), _blockdiag(a_dst21))

    zo = jnp.zeros((N, MID), _F32)
    z16 = jnp.zeros((N, HP), _F32)
    poA, pdA, poB, pdB = _edge8(srcA, dstA, s1, d1, h1,
                                srcB, dstB, s2, d2, h2, zo, z16)

    expd = jnp.pad(jnp.repeat(jnp.eye(8, dtype=_F32), 8, axis=1),
                   ((0, HP - 8), (0, 0)))
    sd12w = jnp.pad(jnp.concatenate(
        [a_src12.reshape(MID, 1), a_dst12.reshape(MID, 1)], axis=1),
        ((0, 0), (0, 6)))
    sd22w = jnp.pad(jnp.concatenate(
        [a_src22.reshape(MID, 1), a_dst22.reshape(MID, 1)], axis=1),
        ((0, 0), (0, 6)))
    h12, sd12, h22, sd22 = _tc2(
        poA, pdA, poB, pdB, b11.reshape(1, MID), b21.reshape(1, MID),
        Wg1[:MID], Wg1[MID:], expd, W12, sd12w, W22, sd22w)

    zN = jnp.zeros((N,), _F32)
    poA2, pdA2, poB2, pdB2 = _edge1(
        srcA, dstA, sd12[:, 0], sd12[:, 1], h12,
        srcB, dstB, sd22[:, 0], sd22[:, 1], h22, zo, zN)

    return _tc3(poA2, pdA2.reshape(2, N, 1), poB2, pdB2.reshape(2, N, 1),
                b12.reshape(1, MID), b22.reshape(1, MID), Wg2[:MID], Wg2[MID:])


# speedup vs baseline: 75.0962x; 1.0161x over previous
import functools

import jax
import jax.numpy as jnp
from jax import lax
from jax.experimental import pallas as pl
from jax.experimental.pallas import tpu as pltpu
from jax.experimental.pallas import tpu_sc as plsc

N = 10000
E = 320000
DIN = 128
MID = 64
FOUT = 64
NEG = 0.25
HP = 16
RB = 1000
B = 80
NWORK = 32
EPW = E // NWORK
NCHUNK = EPW // B

_F32 = jnp.float32


def _tc1(x, w1, as1, ad1, w2, as2, ad2):
    def body(x_ref, w1_ref, as1_ref, ad1_ref, w2_ref, as2_ref, ad2_ref,
             h1_ref, s1_ref, d1_ref, h2_ref, s2_ref, d2_ref):
        xb = x_ref[...]
        h1 = jnp.dot(xb, w1_ref[...], preferred_element_type=_F32)
        h1_ref[...] = h1
        s1_ref[...] = jnp.dot(h1, as1_ref[...], preferred_element_type=_F32)
        d1_ref[...] = jnp.dot(h1, ad1_ref[...], preferred_element_type=_F32)
        h2 = jnp.dot(xb, w2_ref[...], preferred_element_type=_F32)
        h2_ref[...] = h2
        s2_ref[...] = jnp.dot(h2, as2_ref[...], preferred_element_type=_F32)
        d2_ref[...] = jnp.dot(h2, ad2_ref[...], preferred_element_type=_F32)

    nb = N // RB
    whole = lambda shape: pl.BlockSpec(shape, lambda i: (0,) * len(shape))
    rows = lambda w: pl.BlockSpec((RB, w), lambda i: (i, 0))
    return pl.pallas_call(
        body,
        grid=(nb,),
        in_specs=[rows(DIN), whole((DIN, MID)), whole((MID, HP)), whole((MID, HP)),
                  whole((DIN, MID)), whole((MID, HP)), whole((MID, HP))],
        out_specs=[rows(MID), rows(HP), rows(HP), rows(MID), rows(HP), rows(HP)],
        out_shape=[jax.ShapeDtypeStruct((N, MID), _F32),
                   jax.ShapeDtypeStruct((N, HP), _F32),
                   jax.ShapeDtypeStruct((N, HP), _F32),
                   jax.ShapeDtypeStruct((N, MID), _F32),
                   jax.ShapeDtypeStruct((N, HP), _F32),
                   jax.ShapeDtypeStruct((N, HP), _F32)],
    )(x, w1, as1, ad1, w2, as2, ad2)


def _edge_pair(nhead):
    wide = nhead == 8
    den_shape = (N, HP) if wide else (N,)
    row_shape = (B, HP) if wide else (B,)
    pd_shape = (2, N, HP) if wide else (2, N)

    mesh = plsc.VectorSubcoreMesh(core_axis_name="c", subcore_axis_name="s")
    out_type = (
        jax.ShapeDtypeStruct((2, N, MID), _F32),
        jax.ShapeDtypeStruct(pd_shape, _F32),
        jax.ShapeDtypeStruct((2, N, MID), _F32),
        jax.ShapeDtypeStruct(pd_shape, _F32),
    )
    buf = lambda: [pltpu.VMEM(row_shape, _F32),
                   pltpu.VMEM(row_shape, _F32),
                   pltpu.VMEM(row_shape, _F32),
                   pltpu.VMEM((B, MID), _F32),
                   pltpu.VMEM((B, MID), _F32),
                   pltpu.SemaphoreType.DMA,
                   pltpu.SemaphoreType.DMA]
    scratch = [
        pltpu.VMEM((NCHUNK, B), jnp.int32),
        pltpu.VMEM((NCHUNK, B), jnp.int32),
        *buf(), *buf(),
        pltpu.VMEM_SHARED((N, MID), _F32),
        pltpu.VMEM_SHARED(den_shape, _F32),
    ]

    def body(srcA, dstA, sA, dA, hA, srcB, dstB, sB, dB, hB, zo, zd,
             poA, pdA, poB, pdB,
             ixs, ixd, sr0, dr0, ex0, hr0, ms0, g0, s0, sr1, dr1, ex1, hr1,
             ms1, g1, s1, acc_out, acc_den):
        cid = lax.axis_index("c")
        sid = lax.axis_index("s")
        wid = sid * 2 + cid
        rbase = wid * NCHUNK
        iota = lax.iota(jnp.int32, 16)
        colpat = [lax.shift_right_logical(iota, 3) + 2 * v for v in range(4)]
        bufs = ((sr0, dr0, ex0, hr0, ms0, g0, s0),
                (sr1, dr1, ex1, hr1, ms1, g1, s1))

        for (srcR, dstR, sR, dR, hR, poR, pdR) in (
                (srcA, dstA, sA, dA, hA, poA, pdA),
                (srcB, dstB, sB, dB, hB, poB, pdB)):

            @pl.when(sid == 0)
            def _():
                pltpu.sync_copy(zo, acc_out)
                pltpu.sync_copy(zd, acc_den)
            pltpu.sync_copy(srcR.at[pl.ds(rbase, NCHUNK)], ixs)
            pltpu.sync_copy(dstR.at[pl.ds(rbase, NCHUNK)], ixd)
            plsc.subcore_barrier()

            def fire(i, bf):
                sr, dr, exb, hr, ms, sem, ssem = bf
                pltpu.async_copy(sR.at[ixs.at[i]], sr, sem)
                pltpu.async_copy(dR.at[ixd.at[i]], dr, sem)
                pltpu.async_copy(hR.at[ixs.at[i]], hr, sem)

            def waitg(i, bf):
                sr, dr, exb, hr, ms, sem, ssem = bf
                pltpu.make_async_copy(sR.at[ixs.at[i]], sr, sem).wait()
                pltpu.make_async_copy(dR.at[ixd.at[i]], dr, sem).wait()
                pltpu.make_async_copy(hR.at[ixs.at[i]], hr, sem).wait()

            def drain_scatter(i, bf):
                sr, dr, exb, hr, ms, sem, ssem = bf
                pltpu.make_async_copy(exb, acc_den.at[ixd.at[i]], ssem).wait()
                pltpu.make_async_copy(ms, acc_out.at[ixd.at[i]], ssem).wait()

            def compute(i, bf, first):
                sr, dr, exb, hr, ms, sem, ssem = bf
                if not first:
                    drain_scatter(i, bf)
                if wide:
                    def erow(b, c):
                        ev = sr[b] + dr[b]
                        ev = jnp.where(ev >= 0.0, ev, NEG * ev)
                        exv = jnp.exp(ev)
                        exb[b] = exv
                        for v in range(4):
                            sl = pl.ds(v * 16, 16)
                            ms[b, sl] = hr[b, sl] * exv[colpat[v]]
                        return c
                    lax.fori_loop(0, B, erow, 0)
                else:
                    def erow(k, c):
                        sl = pl.ds(k * 16, 16)
                        ev = sr[sl] + dr[sl]
                        ev = jnp.where(ev >= 0.0, ev, NEG * ev)
                        exb[sl] = jnp.exp(ev)
                        return c
                    lax.fori_loop(0, B // 16, erow, 0)

                    def mrow(b, c):
                        exv = plsc.load_gather(
                            exb, [jnp.full((16,), b, jnp.int32)])
                        for v in range(4):
                            sl = pl.ds(v * 16, 16)
                            ms[b, sl] = hr[b, sl] * exv
                        return c
                    lax.fori_loop(0, B, mrow, 0)

                pltpu.async_copy(exb, acc_den.at[ixd.at[i]], ssem, add=True)
                pltpu.async_copy(ms, acc_out.at[ixd.at[i]], ssem, add=True)

            fire(0, bufs[0])
            fire(1, bufs[1])
            waitg(0, bufs[0])
            compute(0, bufs[0], True)
            fire(2, bufs[0])
            waitg(1, bufs[1])
            compute(1, bufs[1], True)

            def step(k, c):
                ia = 2 * k + 2
                fire(ia + 1, bufs[1])
                waitg(ia, bufs[0])
                compute(ia, bufs[0], False)
                fire(ia + 2, bufs[0])
                waitg(ia + 1, bufs[1])
                compute(ia + 1, bufs[1], False)
                return c

            lax.fori_loop(0, (NCHUNK - 3) // 2, step, 0)
            waitg(NCHUNK - 1, bufs[0])
            compute(NCHUNK - 1, bufs[0], False)
            drain_scatter(NCHUNK - 1, bufs[0])
            drain_scatter(NCHUNK - 2, bufs[1])
            plsc.subcore_barrier()

            @pl.when(sid == 0)
            def _():
                pltpu.sync_copy(acc_out, poR.at[cid])
                pltpu.sync_copy(acc_den, pdR.at[cid])
            plsc.subcore_barrier()

    return pl.kernel(
        body, out_type=out_type, mesh=mesh, scratch_types=scratch,
        compiler_params=pltpu.CompilerParams(
            needs_layout_passes=False, use_tc_tiling_on_sc=False))


_edge8 = _edge_pair(8)
_edge1 = _edge_pair(1)


def _tc2(poA, pdA, poB, pdB, b1, b2, wg1a, wg1b, expd, w12, sd12w, w22, sd22w):
    def body(poA_ref, pdA_ref, poB_ref, pdB_ref, b1_ref, b2_ref, wg1a_ref,
             wg1b_ref, expd_ref, w12_ref, sd12w_ref, w22_ref, sd22w_ref,
             h12_ref, sd12_ref, h22_ref, sd22_ref):
        expd_m = expd_ref[...]
        o1 = poA_ref[0] + poA_ref[1]
        rec1 = 1.0 / (pdA_ref[0] + pdA_ref[1] + 1e-16)
        ns11 = o1 * jnp.dot(rec1, expd_m, preferred_element_type=_F32) + b1_ref[...]
        ns11 = jnp.where(ns11 > 0.0, ns11, jnp.exp(ns11) - 1.0)
        o2 = poB_ref[0] + poB_ref[1]
        rec2 = 1.0 / (pdB_ref[0] + pdB_ref[1] + 1e-16)
        ns21 = o2 * jnp.dot(rec2, expd_m, preferred_element_type=_F32) + b2_ref[...]
        ns21 = jnp.where(ns21 > 0.0, ns21, jnp.exp(ns21) - 1.0)
        zl = (jnp.dot(ns11, wg1a_ref[...], preferred_element_type=_F32) +
              jnp.dot(ns21, wg1b_ref[...], preferred_element_type=_F32))
        z = 1.0 / (1.0 + jnp.exp(-zl))
        midv = z * ns11 + (1.0 - z) * ns21
        h12 = jnp.dot(midv, w12_ref[...], preferred_element_type=_F32)
        h12_ref[...] = h12
        sd12_ref[...] = jnp.dot(h12, sd12w_ref[...], preferred_element_type=_F32)
        h22 = jnp.dot(midv, w22_ref[...], preferred_element_type=_F32)
        h22_ref[...] = h22
        sd22_ref[...] = jnp.dot(h22, sd22w_ref[...], preferred_element_type=_F32)

    nb = N // RB
    whole = lambda shape: pl.BlockSpec(shape, lambda i: (0,) * len(shape))
    rows = lambda w: pl.BlockSpec((RB, w), lambda i: (i, 0))
    prow = lambda w: pl.BlockSpec((2, RB, w), lambda i: (0, i, 0))
    return pl.pallas_call(
        body,
        grid=(nb,),
        in_specs=[prow(MID), prow(HP), prow(MID), prow(HP),
                  whole((1, MID)), whole((1, MID)),
                  whole((MID, MID)), whole((MID, MID)), whole((HP, MID)),
                  whole((MID, MID)), whole((MID, 8)),
                  whole((MID, MID)), whole((MID, 8))],
        out_specs=[rows(MID), rows(8), rows(MID), rows(8)],
        out_shape=[jax.ShapeDtypeStruct((N, MID), _F32),
                   jax.ShapeDtypeStruct((N, 8), _F32),
                   jax.ShapeDtypeStruct((N, MID), _F32),
                   jax.ShapeDtypeStruct((N, 8), _F32)],
    )(poA, pdA, poB, pdB, b1, b2, wg1a, wg1b, expd, w12, sd12w, w22, sd22w)


def _tc3(poA, pdA, poB, pdB, b1, b2, wg2a, wg2b):
    def body(poA_ref, pdA_ref, poB_ref, pdB_ref, b1_ref, b2_ref,
             wg2a_ref, wg2b_ref, out_ref):
        o1 = poA_ref[0] + poA_ref[1]
        den1 = pdA_ref[0] + pdA_ref[1]
        ns12 = o1 * (1.0 / (den1 + 1e-16)) + b1_ref[...]
        o2 = poB_ref[0] + poB_ref[1]
        den2 = pdB_ref[0] + pdB_ref[1]
        ns22 = o2 * (1.0 / (den2 + 1e-16)) + b2_ref[...]
        zl = (jnp.dot(ns12, wg2a_ref[...], preferred_element_type=_F32) +
              jnp.dot(ns22, wg2b_ref[...], preferred_element_type=_F32))
        z = 1.0 / (1.0 + jnp.exp(-zl))
        outv = z * ns12 + (1.0 - z) * ns22
        m = jnp.max(outv, axis=1, keepdims=True)
        sh = outv - m
        lse = jnp.log(jnp.sum(jnp.exp(sh), axis=1, keepdims=True))
        out_ref[...] = sh - lse

    nb = N // RB
    whole = lambda shape: pl.BlockSpec(shape, lambda i: (0,) * len(shape))
    rows = lambda w: pl.BlockSpec((RB, w), lambda i: (i, 0))
    prow = lambda w: pl.BlockSpec((2, RB, w), lambda i: (0, i, 0))
    return pl.pallas_call(
        body,
        grid=(nb,),
        in_specs=[prow(MID), prow(1), prow(MID), prow(1),
                  whole((1, MID)), whole((1, MID)),
                  whole((MID, MID)), whole((MID, MID))],
        out_specs=rows(MID),
        out_shape=jax.ShapeDtypeStruct((N, MID), _F32),
    )(poA, pdA, poB, pdB, b1, b2, wg2a, wg2b)


def _blockdiag(a):
    h, c = a.shape
    eye = jnp.eye(h, dtype=_F32)
    m = (a[:, :, None] * eye[:, None, :]).reshape(h * c, h)
    return jnp.pad(m, ((0, 0), (0, HP - h)))


def kernel(node_feature, one_adj_list, two_adj_list, W11, a_src11, a_dst11,
           b11, W21, a_src21, a_dst21, b21, Wg1, W12, a_src12, a_dst12, b12,
           W22, a_src22, a_dst22, b22, Wg2):
    srcA = one_adj_list[0].reshape(E // B, B)
    dstA = one_adj_list[1].reshape(E // B, B)
    srcB = two_adj_list[0].reshape(E // B, B)
    dstB = two_adj_list[1].reshape(E // B, B)

    h1, s1, d1, h2, s2, d2 = _tc1(
        node_feature, W11, _blockdiag(a_src11), _blockdiag(a_dst11),
        W21, _blockdiag(a_src21), _blockdiag(a_dst21))

    zo = jnp.zeros((N, MID), _F32)
    z16 = jnp.zeros((N, HP), _F32)
    poA, pdA, poB, pdB = _edge8(srcA, dstA, s1, d1, h1,
                                srcB, dstB, s2, d2, h2, zo, z16)

    expd = jnp.pad(jnp.repeat(jnp.eye(8, dtype=_F32), 8, axis=1),
                   ((0, HP - 8), (0, 0)))
    sd12w = jnp.pad(jnp.concatenate(
        [a_src12.reshape(MID, 1), a_dst12.reshape(MID, 1)], axis=1),
        ((0, 0), (0, 6)))
    sd22w = jnp.pad(jnp.concatenate(
        [a_src22.reshape(MID, 1), a_dst22.reshape(MID, 1)], axis=1),
        ((0, 0), (0, 6)))
    h12, sd12, h22, sd22 = _tc2(
        poA, pdA, poB, pdB, b11.reshape(1, MID), b21.reshape(1, MID),
        Wg1[:MID], Wg1[MID:], expd, W12, sd12w, W22, sd22w)

    zN = jnp.zeros((N,), _F32)
    poA2, pdA2, poB2, pdB2 = _edge1(
        srcA, dstA, sd12[:, 0], sd12[:, 1], h12,
        srcB, dstB, sd22[:, 0], sd22[:, 1], h22, zo, zN)

    return _tc3(poA2, pdA2.reshape(2, N, 1), poB2, pdB2.reshape(2, N, 1),
                b12.reshape(1, MID), b22.reshape(1, MID), Wg2[:MID], Wg2[MID:])


# speedup vs baseline: 166.9868x; 2.2236x over previous
import functools

import jax
import jax.numpy as jnp
from jax import lax
from jax.experimental import pallas as pl
from jax.experimental.pallas import tpu as pltpu
from jax.experimental.pallas import tpu_sc as plsc

N = 10000
E = 320000
DIN = 128
MID = 64
FOUT = 64
NEG = 0.25
HP = 16
UW = MID + HP
RB = 1000
B = 80
NWORK = 32
EPW = E // NWORK
NCHUNK = EPW // B

_F32 = jnp.float32


def _tc1(x, w1, as1, ad1, w2, as2, ad2):
    def body(x_ref, w1_ref, as1_ref, ad1_ref, w2_ref, as2_ref, ad2_ref,
             u1_ref, d1_ref, u2_ref, d2_ref):
        xb = x_ref[...]
        h1 = jnp.dot(xb, w1_ref[...], preferred_element_type=_F32)
        s1 = jnp.dot(h1, as1_ref[...], preferred_element_type=_F32)
        u1_ref[...] = jnp.concatenate([h1, s1], axis=1)
        d1_ref[...] = jnp.dot(h1, ad1_ref[...], preferred_element_type=_F32)
        h2 = jnp.dot(xb, w2_ref[...], preferred_element_type=_F32)
        s2 = jnp.dot(h2, as2_ref[...], preferred_element_type=_F32)
        u2_ref[...] = jnp.concatenate([h2, s2], axis=1)
        d2_ref[...] = jnp.dot(h2, ad2_ref[...], preferred_element_type=_F32)

    nb = N // RB
    whole = lambda shape: pl.BlockSpec(shape, lambda i: (0,) * len(shape))
    rows = lambda w: pl.BlockSpec((RB, w), lambda i: (i, 0))
    return pl.pallas_call(
        body,
        grid=(nb,),
        in_specs=[rows(DIN), whole((DIN, MID)), whole((MID, HP)), whole((MID, HP)),
                  whole((DIN, MID)), whole((MID, HP)), whole((MID, HP))],
        out_specs=[rows(UW), rows(HP), rows(UW), rows(HP)],
        out_shape=[jax.ShapeDtypeStruct((N, UW), _F32),
                   jax.ShapeDtypeStruct((N, HP), _F32),
                   jax.ShapeDtypeStruct((N, UW), _F32),
                   jax.ShapeDtypeStruct((N, HP), _F32)],
    )(x, w1, as1, ad1, w2, as2, ad2)


def _edge_pair(nhead):
    wide = nhead == 8
    dt_shape = (N, HP) if wide else (N,)
    dr_shape = (B, HP) if wide else (B,)

    mesh = plsc.VectorSubcoreMesh(core_axis_name="c", subcore_axis_name="s")
    out_type = (
        jax.ShapeDtypeStruct((2, N, UW), _F32),
        jax.ShapeDtypeStruct((2, N, UW), _F32),
    )
    buf = lambda: [pltpu.VMEM((B, UW), _F32),
                   pltpu.VMEM(dr_shape, _F32),
                   pltpu.VMEM((B, UW), _F32),
                   pltpu.SemaphoreType.DMA,
                   pltpu.SemaphoreType.DMA]
    scratch = [
        pltpu.VMEM((NCHUNK, B), jnp.int32),
        pltpu.VMEM((NCHUNK, B), jnp.int32),
        *buf(), *buf(),
        pltpu.VMEM_SHARED((N, UW), _F32),
    ]

    def body(srcA, dstA, uA, dA, srcB, dstB, uB, dB, zo,
             poA, poB,
             ixs, ixd, ur0, dr0, ms0, g0, s0, ur1, dr1, ms1, g1, s1,
             acc):
        cid = lax.axis_index("c")
        sid = lax.axis_index("s")
        wid = sid * 2 + cid
        rbase = wid * NCHUNK
        iota = lax.iota(jnp.int32, 16)
        colpat = [lax.shift_right_logical(iota, 3) + 2 * v for v in range(4)]
        bufs = ((ur0, dr0, ms0, g0, s0), (ur1, dr1, ms1, g1, s1))

        for (srcR, dstR, uR, dR, poR) in (
                (srcA, dstA, uA, dA, poA),
                (srcB, dstB, uB, dB, poB)):

            @pl.when(sid == 0)
            def _():
                pltpu.sync_copy(zo, acc)
            pltpu.sync_copy(srcR.at[pl.ds(rbase, NCHUNK)], ixs)
            pltpu.sync_copy(dstR.at[pl.ds(rbase, NCHUNK)], ixd)
            plsc.subcore_barrier()

            def fire(i, bf):
                ur, dr, ms, sem, ssem = bf
                pltpu.async_copy(uR.at[ixs.at[i]], ur, sem)
                pltpu.async_copy(dR.at[ixd.at[i]], dr, sem)

            def waitg(i, bf):
                ur, dr, ms, sem, ssem = bf
                pltpu.make_async_copy(uR.at[ixs.at[i]], ur, sem).wait()
                pltpu.make_async_copy(dR.at[ixd.at[i]], dr, sem).wait()

            def drain_scatter(i, bf):
                ur, dr, ms, sem, ssem = bf
                pltpu.make_async_copy(ms, acc.at[ixd.at[i]], ssem).wait()

            def compute(i, bf, first):
                ur, dr, ms, sem, ssem = bf
                if not first:
                    drain_scatter(i, bf)

                @plsc.parallel_loop(0, B, unroll=4)
                def _(b):
                    if wide:
                        sv = ur[b, pl.ds(MID, HP)]
                        dv = dr[b]
                    else:
                        sv = plsc.load_gather(
                            ur, [jnp.full((16,), b, jnp.int32),
                                 jnp.full((16,), MID, jnp.int32)])
                        dv = plsc.load_gather(
                            dr, [jnp.full((16,), b, jnp.int32)])
                    ev = sv + dv
                    ev = jnp.where(ev >= 0.0, ev, NEG * ev)
                    exv = jnp.exp(ev)
                    ms[b, pl.ds(MID, HP)] = exv
                    for v in range(4):
                        sl = pl.ds(v * 16, 16)
                        if wide:
                            ms[b, sl] = ur[b, sl] * exv[colpat[v]]
                        else:
                            ms[b, sl] = ur[b, sl] * exv

                pltpu.async_copy(ms, acc.at[ixd.at[i]], ssem, add=True)

            fire(0, bufs[0])
            fire(1, bufs[1])
            waitg(0, bufs[0])
            compute(0, bufs[0], True)
            fire(2, bufs[0])
            waitg(1, bufs[1])
            compute(1, bufs[1], True)

            def step(k, c):
                ia = 2 * k + 2
                fire(ia + 1, bufs[1])
                waitg(ia, bufs[0])
                compute(ia, bufs[0], False)
                fire(ia + 2, bufs[0])
                waitg(ia + 1, bufs[1])
                compute(ia + 1, bufs[1], False)
                return c

            lax.fori_loop(0, (NCHUNK - 3) // 2, step, 0)
            waitg(NCHUNK - 1, bufs[0])
            compute(NCHUNK - 1, bufs[0], False)
            drain_scatter(NCHUNK - 1, bufs[0])
            drain_scatter(NCHUNK - 2, bufs[1])
            plsc.subcore_barrier()

            @pl.when(sid == 0)
            def _():
                pltpu.sync_copy(acc, poR.at[cid])
            plsc.subcore_barrier()

    return pl.kernel(
        body, out_type=out_type, mesh=mesh, scratch_types=scratch,
        compiler_params=pltpu.CompilerParams(
            needs_layout_passes=False, use_tc_tiling_on_sc=False))


_edge8 = _edge_pair(8)
_edge1 = _edge_pair(1)


def _tc2(poA, poB, b1, b2, wg1a, wg1b, expd, w12, as12r, sd12w, w22, as22r,
         sd22w):
    def body(poA_ref, poB_ref, b1_ref, b2_ref, wg1a_ref,
             wg1b_ref, expd_ref, w12_ref, as12r_ref, sd12w_ref, w22_ref,
             as22r_ref, sd22w_ref, u12_ref, d12_ref, u22_ref, d22_ref):
        expd_m = expd_ref[...]
        pA = poA_ref[0] + poA_ref[1]
        o1 = pA[:, :MID]
        rec1 = 1.0 / (pA[:, MID:] + 1e-16)
        ns11 = o1 * jnp.dot(rec1, expd_m, preferred_element_type=_F32) + b1_ref[...]
        ns11 = jnp.where(ns11 > 0.0, ns11, jnp.exp(ns11) - 1.0)
        pB = poB_ref[0] + poB_ref[1]
        o2 = pB[:, :MID]
        rec2 = 1.0 / (pB[:, MID:] + 1e-16)
        ns21 = o2 * jnp.dot(rec2, expd_m, preferred_element_type=_F32) + b2_ref[...]
        ns21 = jnp.where(ns21 > 0.0, ns21, jnp.exp(ns21) - 1.0)
        zl = (jnp.dot(ns11, wg1a_ref[...], preferred_element_type=_F32) +
              jnp.dot(ns21, wg1b_ref[...], preferred_element_type=_F32))
        z = 1.0 / (1.0 + jnp.exp(-zl))
        midv = z * ns11 + (1.0 - z) * ns21
        h12 = jnp.dot(midv, w12_ref[...], preferred_element_type=_F32)
        s12 = jnp.dot(h12, as12r_ref[...], preferred_element_type=_F32)
        u12_ref[...] = jnp.concatenate([h12, s12], axis=1)
        d12_ref[...] = jnp.dot(h12, sd12w_ref[...], preferred_element_type=_F32)
        h22 = jnp.dot(midv, w22_ref[...], preferred_element_type=_F32)
        s22 = jnp.dot(h22, as22r_ref[...], preferred_element_type=_F32)
        u22_ref[...] = jnp.concatenate([h22, s22], axis=1)
        d22_ref[...] = jnp.dot(h22, sd22w_ref[...], preferred_element_type=_F32)

    nb = N // RB
    whole = lambda shape: pl.BlockSpec(shape, lambda i: (0,) * len(shape))
    rows = lambda w: pl.BlockSpec((RB, w), lambda i: (i, 0))
    prow = lambda w: pl.BlockSpec((2, RB, w), lambda i: (0, i, 0))
    return pl.pallas_call(
        body,
        grid=(nb,),
        in_specs=[prow(UW), prow(UW),
                  whole((1, MID)), whole((1, MID)),
                  whole((MID, MID)), whole((MID, MID)), whole((HP, MID)),
                  whole((MID, MID)), whole((MID, HP)), whole((MID, 8)),
                  whole((MID, MID)), whole((MID, HP)), whole((MID, 8))],
        out_specs=[rows(UW), rows(8), rows(UW), rows(8)],
        out_shape=[jax.ShapeDtypeStruct((N, UW), _F32),
                   jax.ShapeDtypeStruct((N, 8), _F32),
                   jax.ShapeDtypeStruct((N, UW), _F32),
                   jax.ShapeDtypeStruct((N, 8), _F32)],
    )(poA, poB, b1, b2, wg1a, wg1b, expd, w12, as12r, sd12w, w22, as22r,
      sd22w)


def _tc3(poA, poB, b1, b2, wg2a, wg2b):
    def body(poA_ref, poB_ref, b1_ref, b2_ref, wg2a_ref, wg2b_ref, out_ref):
        pA = poA_ref[0] + poA_ref[1]
        ns12 = pA[:, :MID] * (1.0 / (pA[:, MID:MID + 1] + 1e-16)) + b1_ref[...]
        pB = poB_ref[0] + poB_ref[1]
        ns22 = pB[:, :MID] * (1.0 / (pB[:, MID:MID + 1] + 1e-16)) + b2_ref[...]
        zl = (jnp.dot(ns12, wg2a_ref[...], preferred_element_type=_F32) +
              jnp.dot(ns22, wg2b_ref[...], preferred_element_type=_F32))
        z = 1.0 / (1.0 + jnp.exp(-zl))
        outv = z * ns12 + (1.0 - z) * ns22
        m = jnp.max(outv, axis=1, keepdims=True)
        sh = outv - m
        lse = jnp.log(jnp.sum(jnp.exp(sh), axis=1, keepdims=True))
        out_ref[...] = sh - lse

    nb = N // RB
    whole = lambda shape: pl.BlockSpec(shape, lambda i: (0,) * len(shape))
    rows = lambda w: pl.BlockSpec((RB, w), lambda i: (i, 0))
    prow = lambda w: pl.BlockSpec((2, RB, w), lambda i: (0, i, 0))
    return pl.pallas_call(
        body,
        grid=(nb,),
        in_specs=[prow(UW), prow(UW),
                  whole((1, MID)), whole((1, MID)),
                  whole((MID, MID)), whole((MID, MID))],
        out_specs=rows(MID),
        out_shape=jax.ShapeDtypeStruct((N, MID), _F32),
    )(poA, poB, b1, b2, wg2a, wg2b)


def _blockdiag(a):
    h, c = a.shape
    eye = jnp.eye(h, dtype=_F32)
    m = (a[:, :, None] * eye[:, None, :]).reshape(h * c, h)
    return jnp.pad(m, ((0, 0), (0, HP - h)))


def kernel(node_feature, one_adj_list, two_adj_list, W11, a_src11, a_dst11,
           b11, W21, a_src21, a_dst21, b21, Wg1, W12, a_src12, a_dst12, b12,
           W22, a_src22, a_dst22, b22, Wg2):
    srcA = one_adj_list[0].reshape(E // B, B)
    dstA = one_adj_list[1].reshape(E // B, B)
    srcB = two_adj_list[0].reshape(E // B, B)
    dstB = two_adj_list[1].reshape(E // B, B)

    u1, d1, u2, d2 = _tc1(
        node_feature, W11, _blockdiag(a_src11), _blockdiag(a_dst11),
        W21, _blockdiag(a_src21), _blockdiag(a_dst21))

    zo = jnp.zeros((N, UW), _F32)
    poA, poB = _edge8(srcA, dstA, u1, d1, srcB, dstB, u2, d2, zo)

    expd = jnp.pad(jnp.repeat(jnp.eye(8, dtype=_F32), 8, axis=1),
                   ((0, HP - 8), (0, 0)))
    as12r = jnp.tile(a_src12.reshape(MID, 1), (1, HP))
    as22r = jnp.tile(a_src22.reshape(MID, 1), (1, HP))
    sd12w = jnp.pad(a_dst12.reshape(MID, 1), ((0, 0), (0, 7)))
    sd22w = jnp.pad(a_dst22.reshape(MID, 1), ((0, 0), (0, 7)))
    u12, d12, u22, d22 = _tc2(
        poA, poB, b11.reshape(1, MID), b21.reshape(1, MID),
        Wg1[:MID], Wg1[MID:], expd, W12, as12r, sd12w, W22, as22r, sd22w)

    poA2, poB2 = _edge1(srcA, dstA, u12, d12[:, 0],
                        srcB, dstB, u22, d22[:, 0], zo)

    return _tc3(poA2, poB2, b12.reshape(1, MID), b22.reshape(1, MID),
                Wg2[:MID], Wg2[MID:])


# speedup vs baseline: 169.8509x; 1.0172x over previous
import functools

import jax
import jax.numpy as jnp
from jax import lax
from jax.experimental import pallas as pl
from jax.experimental.pallas import tpu as pltpu
from jax.experimental.pallas import tpu_sc as plsc

N = 10000
E = 320000
DIN = 128
MID = 64
FOUT = 64
NEG = 0.25
HP = 16
UW = MID + HP
RB = 2000
B = 80
NWORK = 32
EPW = E // NWORK
NCHUNK = EPW // B

_F32 = jnp.float32


def _tc1(x, w1, as1, ad1, w2, as2, ad2):
    def body(x_ref, w1_ref, as1_ref, ad1_ref, w2_ref, as2_ref, ad2_ref,
             u1_ref, d1_ref, u2_ref, d2_ref):
        xb = x_ref[...]
        h1 = jnp.dot(xb, w1_ref[...], preferred_element_type=_F32)
        s1 = jnp.dot(h1, as1_ref[...], preferred_element_type=_F32)
        u1_ref[...] = jnp.concatenate([h1, s1], axis=1)
        d1_ref[...] = jnp.dot(h1, ad1_ref[...], preferred_element_type=_F32)
        h2 = jnp.dot(xb, w2_ref[...], preferred_element_type=_F32)
        s2 = jnp.dot(h2, as2_ref[...], preferred_element_type=_F32)
        u2_ref[...] = jnp.concatenate([h2, s2], axis=1)
        d2_ref[...] = jnp.dot(h2, ad2_ref[...], preferred_element_type=_F32)

    nb = N // RB
    whole = lambda shape: pl.BlockSpec(shape, lambda i: (0,) * len(shape))
    rows = lambda w: pl.BlockSpec((RB, w), lambda i: (i, 0))
    return pl.pallas_call(
        body,
        grid=(nb,),
        in_specs=[rows(DIN), whole((DIN, MID)), whole((MID, HP)), whole((MID, HP)),
                  whole((DIN, MID)), whole((MID, HP)), whole((MID, HP))],
        out_specs=[rows(UW), rows(HP), rows(UW), rows(HP)],
        out_shape=[jax.ShapeDtypeStruct((N, UW), _F32),
                   jax.ShapeDtypeStruct((N, HP), _F32),
                   jax.ShapeDtypeStruct((N, UW), _F32),
                   jax.ShapeDtypeStruct((N, HP), _F32)],
    )(x, w1, as1, ad1, w2, as2, ad2)


def _edge_pair(nhead):
    wide = nhead == 8
    dt_shape = (N, HP) if wide else (N,)
    dr_shape = (B, HP) if wide else (B,)

    mesh = plsc.VectorSubcoreMesh(core_axis_name="c", subcore_axis_name="s")
    out_type = (
        jax.ShapeDtypeStruct((2, N, UW), _F32),
        jax.ShapeDtypeStruct((2, N, UW), _F32),
    )
    buf = lambda: [pltpu.VMEM((B, UW), _F32),
                   pltpu.VMEM(dr_shape, _F32),
                   pltpu.VMEM((B, UW), _F32),
                   pltpu.SemaphoreType.DMA,
                   pltpu.SemaphoreType.DMA]
    scratch = [
        pltpu.VMEM((NCHUNK, B), jnp.int32),
        pltpu.VMEM((NCHUNK, B), jnp.int32),
        *buf(), *buf(),
        pltpu.VMEM_SHARED((N, UW), _F32),
    ]

    def body(srcA, dstA, uA, dA, srcB, dstB, uB, dB, zo,
             poA, poB,
             ixs, ixd, ur0, dr0, ms0, g0, s0, ur1, dr1, ms1, g1, s1,
             acc):
        cid = lax.axis_index("c")
        sid = lax.axis_index("s")
        wid = sid * 2 + cid
        rbase = wid * NCHUNK
        iota = lax.iota(jnp.int32, 16)
        colpat = [lax.shift_right_logical(iota, 3) + 2 * v for v in range(4)]
        bufs = ((ur0, dr0, ms0, g0, s0), (ur1, dr1, ms1, g1, s1))

        for (srcR, dstR, uR, dR, poR) in (
                (srcA, dstA, uA, dA, poA),
                (srcB, dstB, uB, dB, poB)):

            @pl.when(sid == 0)
            def _():
                pltpu.sync_copy(zo, acc)
            pltpu.sync_copy(srcR.at[pl.ds(rbase, NCHUNK)], ixs)
            pltpu.sync_copy(dstR.at[pl.ds(rbase, NCHUNK)], ixd)
            plsc.subcore_barrier()

            def fire(i, bf):
                ur, dr, ms, sem, ssem = bf
                pltpu.async_copy(uR.at[ixs.at[i]], ur, sem)
                pltpu.async_copy(dR.at[ixd.at[i]], dr, sem)

            def waitg(i, bf):
                ur, dr, ms, sem, ssem = bf
                pltpu.make_async_copy(uR.at[ixs.at[i]], ur, sem).wait()
                pltpu.make_async_copy(dR.at[ixd.at[i]], dr, sem).wait()

            def drain_scatter(i, bf):
                ur, dr, ms, sem, ssem = bf
                pltpu.make_async_copy(ms, acc.at[ixd.at[i]], ssem).wait()

            def compute(i, bf, first):
                ur, dr, ms, sem, ssem = bf
                if not first:
                    drain_scatter(i, bf)

                @plsc.parallel_loop(0, B, unroll=8)
                def _(b):
                    if wide:
                        sv = ur[b, pl.ds(MID, HP)]
                        dv = dr[b]
                    else:
                        sv = plsc.load_gather(
                            ur, [jnp.full((16,), b, jnp.int32),
                                 jnp.full((16,), MID, jnp.int32)])
                        dv = plsc.load_gather(
                            dr, [jnp.full((16,), b, jnp.int32)])
                    ev = sv + dv
                    ev = jnp.where(ev >= 0.0, ev, NEG * ev)
                    exv = jnp.exp(ev)
                    ms[b, pl.ds(MID, HP)] = exv
                    for v in range(4):
                        sl = pl.ds(v * 16, 16)
                        if wide:
                            ms[b, sl] = ur[b, sl] * exv[colpat[v]]
                        else:
                            ms[b, sl] = ur[b, sl] * exv

                pltpu.async_copy(ms, acc.at[ixd.at[i]], ssem, add=True)

            fire(0, bufs[0])
            fire(1, bufs[1])
            waitg(0, bufs[0])
            compute(0, bufs[0], True)
            fire(2, bufs[0])
            waitg(1, bufs[1])
            compute(1, bufs[1], True)

            def step(k, c):
                ia = 2 * k + 2
                fire(ia + 1, bufs[1])
                waitg(ia, bufs[0])
                compute(ia, bufs[0], False)
                fire(ia + 2, bufs[0])
                waitg(ia + 1, bufs[1])
                compute(ia + 1, bufs[1], False)
                return c

            lax.fori_loop(0, (NCHUNK - 3) // 2, step, 0)
            waitg(NCHUNK - 1, bufs[0])
            compute(NCHUNK - 1, bufs[0], False)
            drain_scatter(NCHUNK - 1, bufs[0])
            drain_scatter(NCHUNK - 2, bufs[1])
            plsc.subcore_barrier()

            @pl.when(sid == 0)
            def _():
                pltpu.sync_copy(acc, poR.at[cid])
            plsc.subcore_barrier()

    return pl.kernel(
        body, out_type=out_type, mesh=mesh, scratch_types=scratch,
        compiler_params=pltpu.CompilerParams(
            needs_layout_passes=False, use_tc_tiling_on_sc=False))


_edge8 = _edge_pair(8)
_edge1 = _edge_pair(1)


def _tc2(poA, poB, b1, b2, wg1a, wg1b, expd, w12, as12r, sd12w, w22, as22r,
         sd22w):
    def body(poA_ref, poB_ref, b1_ref, b2_ref, wg1a_ref,
             wg1b_ref, expd_ref, w12_ref, as12r_ref, sd12w_ref, w22_ref,
             as22r_ref, sd22w_ref, u12_ref, d12_ref, u22_ref, d22_ref):
        expd_m = expd_ref[...]
        pA = poA_ref[0] + poA_ref[1]
        o1 = pA[:, :MID]
        rec1 = 1.0 / (pA[:, MID:] + 1e-16)
        ns11 = o1 * jnp.dot(rec1, expd_m, preferred_element_type=_F32) + b1_ref[...]
        ns11 = jnp.where(ns11 > 0.0, ns11, jnp.exp(ns11) - 1.0)
        pB = poB_ref[0] + poB_ref[1]
        o2 = pB[:, :MID]
        rec2 = 1.0 / (pB[:, MID:] + 1e-16)
        ns21 = o2 * jnp.dot(rec2, expd_m, preferred_element_type=_F32) + b2_ref[...]
        ns21 = jnp.where(ns21 > 0.0, ns21, jnp.exp(ns21) - 1.0)
        zl = (jnp.dot(ns11, wg1a_ref[...], preferred_element_type=_F32) +
              jnp.dot(ns21, wg1b_ref[...], preferred_element_type=_F32))
        z = 1.0 / (1.0 + jnp.exp(-zl))
        midv = z * ns11 + (1.0 - z) * ns21
        h12 = jnp.dot(midv, w12_ref[...], preferred_element_type=_F32)
        s12 = jnp.dot(h12, as12r_ref[...], preferred_element_type=_F32)
        u12_ref[...] = jnp.concatenate([h12, s12], axis=1)
        d12_ref[...] = jnp.dot(h12, sd12w_ref[...], preferred_element_type=_F32)
        h22 = jnp.dot(midv, w22_ref[...], preferred_element_type=_F32)
        s22 = jnp.dot(h22, as22r_ref[...], preferred_element_type=_F32)
        u22_ref[...] = jnp.concatenate([h22, s22], axis=1)
        d22_ref[...] = jnp.dot(h22, sd22w_ref[...], preferred_element_type=_F32)

    nb = N // RB
    whole = lambda shape: pl.BlockSpec(shape, lambda i: (0,) * len(shape))
    rows = lambda w: pl.BlockSpec((RB, w), lambda i: (i, 0))
    prow = lambda w: pl.BlockSpec((2, RB, w), lambda i: (0, i, 0))
    return pl.pallas_call(
        body,
        grid=(nb,),
        in_specs=[prow(UW), prow(UW),
                  whole((1, MID)), whole((1, MID)),
                  whole((MID, MID)), whole((MID, MID)), whole((HP, MID)),
                  whole((MID, MID)), whole((MID, HP)), whole((MID, 8)),
                  whole((MID, MID)), whole((MID, HP)), whole((MID, 8))],
        out_specs=[rows(UW), rows(8), rows(UW), rows(8)],
        out_shape=[jax.ShapeDtypeStruct((N, UW), _F32),
                   jax.ShapeDtypeStruct((N, 8), _F32),
                   jax.ShapeDtypeStruct((N, UW), _F32),
                   jax.ShapeDtypeStruct((N, 8), _F32)],
    )(poA, poB, b1, b2, wg1a, wg1b, expd, w12, as12r, sd12w, w22, as22r,
      sd22w)


def _tc3(poA, poB, b1, b2, wg2a, wg2b):
    def body(poA_ref, poB_ref, b1_ref, b2_ref, wg2a_ref, wg2b_ref, out_ref):
        pA = poA_ref[0] + poA_ref[1]
        ns12 = pA[:, :MID] * (1.0 / (pA[:, MID:MID + 1] + 1e-16)) + b1_ref[...]
        pB = poB_ref[0] + poB_ref[1]
        ns22 = pB[:, :MID] * (1.0 / (pB[:, MID:MID + 1] + 1e-16)) + b2_ref[...]
        zl = (jnp.dot(ns12, wg2a_ref[...], preferred_element_type=_F32) +
              jnp.dot(ns22, wg2b_ref[...], preferred_element_type=_F32))
        z = 1.0 / (1.0 + jnp.exp(-zl))
        outv = z * ns12 + (1.0 - z) * ns22
        m = jnp.max(outv, axis=1, keepdims=True)
        sh = outv - m
        lse = jnp.log(jnp.sum(jnp.exp(sh), axis=1, keepdims=True))
        out_ref[...] = sh - lse

    nb = N // RB
    whole = lambda shape: pl.BlockSpec(shape, lambda i: (0,) * len(shape))
    rows = lambda w: pl.BlockSpec((RB, w), lambda i: (i, 0))
    prow = lambda w: pl.BlockSpec((2, RB, w), lambda i: (0, i, 0))
    return pl.pallas_call(
        body,
        grid=(nb,),
        in_specs=[prow(UW), prow(UW),
                  whole((1, MID)), whole((1, MID)),
                  whole((MID, MID)), whole((MID, MID))],
        out_specs=rows(MID),
        out_shape=jax.ShapeDtypeStruct((N, MID), _F32),
    )(poA, poB, b1, b2, wg2a, wg2b)


def _blockdiag(a):
    h, c = a.shape
    eye = jnp.eye(h, dtype=_F32)
    m = (a[:, :, None] * eye[:, None, :]).reshape(h * c, h)
    return jnp.pad(m, ((0, 0), (0, HP - h)))


def kernel(node_feature, one_adj_list, two_adj_list, W11, a_src11, a_dst11,
           b11, W21, a_src21, a_dst21, b21, Wg1, W12, a_src12, a_dst12, b12,
           W22, a_src22, a_dst22, b22, Wg2):
    srcA = one_adj_list[0].reshape(E // B, B)
    dstA = one_adj_list[1].reshape(E // B, B)
    srcB = two_adj_list[0].reshape(E // B, B)
    dstB = two_adj_list[1].reshape(E // B, B)

    u1, d1, u2, d2 = _tc1(
        node_feature, W11, _blockdiag(a_src11), _blockdiag(a_dst11),
        W21, _blockdiag(a_src21), _blockdiag(a_dst21))

    zo = jnp.zeros((N, UW), _F32)
    poA, poB = _edge8(srcA, dstA, u1, d1, srcB, dstB, u2, d2, zo)

    expd = jnp.pad(jnp.repeat(jnp.eye(8, dtype=_F32), 8, axis=1),
                   ((0, HP - 8), (0, 0)))
    as12r = jnp.tile(a_src12.reshape(MID, 1), (1, HP))
    as22r = jnp.tile(a_src22.reshape(MID, 1), (1, HP))
    sd12w = jnp.pad(a_dst12.reshape(MID, 1), ((0, 0), (0, 7)))
    sd22w = jnp.pad(a_dst22.reshape(MID, 1), ((0, 0), (0, 7)))
    u12, d12, u22, d22 = _tc2(
        poA, poB, b11.reshape(1, MID), b21.reshape(1, MID),
        Wg1[:MID], Wg1[MID:], expd, W12, as12r, sd12w, W22, as22r, sd22w)

    poA2, poB2 = _edge1(srcA, dstA, u12, d12[:, 0],
                        srcB, dstB, u22, d22[:, 0], zo)

    return _tc3(poA2, poB2, b12.reshape(1, MID), b22.reshape(1, MID),
                Wg2[:MID], Wg2[MID:])


# speedup vs baseline: 190.6820x; 1.1226x over previous
import functools

import jax
import jax.numpy as jnp
from jax import lax
from jax.experimental import pallas as pl
from jax.experimental.pallas import tpu as pltpu
from jax.experimental.pallas import tpu_sc as plsc

N = 10000
E = 320000
DIN = 128
MID = 64
FOUT = 64
NEG = 0.25
HP = 16
UW = MID + HP
RB = 2000
B = 80
NWORK = 32
EPW = E // NWORK
NCHUNK = EPW // B

_F32 = jnp.float32


def _tc1(x, w1, as1, ad1, w2, as2, ad2):
    def body(x_ref, w1_ref, as1_ref, ad1_ref, w2_ref, as2_ref, ad2_ref,
             u1_ref, d1_ref, u2_ref, d2_ref):
        xb = x_ref[...]
        h1 = jnp.dot(xb, w1_ref[...], preferred_element_type=_F32)
        s1 = jnp.dot(h1, as1_ref[...], preferred_element_type=_F32)
        u1_ref[...] = jnp.concatenate([h1, s1], axis=1)
        d1_ref[...] = jnp.dot(h1, ad1_ref[...], preferred_element_type=_F32)
        h2 = jnp.dot(xb, w2_ref[...], preferred_element_type=_F32)
        s2 = jnp.dot(h2, as2_ref[...], preferred_element_type=_F32)
        u2_ref[...] = jnp.concatenate([h2, s2], axis=1)
        d2_ref[...] = jnp.dot(h2, ad2_ref[...], preferred_element_type=_F32)

    nb = N // RB
    whole = lambda shape: pl.BlockSpec(shape, lambda i: (0,) * len(shape))
    rows = lambda w: pl.BlockSpec((RB, w), lambda i: (i, 0))
    return pl.pallas_call(
        body,
        grid=(nb,),
        in_specs=[rows(DIN), whole((DIN, MID)), whole((MID, HP)), whole((MID, HP)),
                  whole((DIN, MID)), whole((MID, HP)), whole((MID, HP))],
        out_specs=[rows(UW), rows(HP), rows(UW), rows(HP)],
        out_shape=[jax.ShapeDtypeStruct((N, UW), _F32),
                   jax.ShapeDtypeStruct((N, HP), _F32),
                   jax.ShapeDtypeStruct((N, UW), _F32),
                   jax.ShapeDtypeStruct((N, HP), _F32)],
    )(x, w1, as1, ad1, w2, as2, ad2)


def _edge_pair(nhead):
    wide = nhead == 8
    dt_shape = (N, HP) if wide else (N,)
    dr_shape = (B, HP) if wide else (B,)

    mesh = plsc.VectorSubcoreMesh(core_axis_name="c", subcore_axis_name="s")
    out_type = (
        jax.ShapeDtypeStruct((2, N, UW), _F32),
        jax.ShapeDtypeStruct((2, N, UW), _F32),
    )
    buf = lambda: [pltpu.VMEM((B, UW), _F32),
                   pltpu.VMEM(dr_shape, _F32),
                   pltpu.VMEM((B, UW), _F32),
                   pltpu.SemaphoreType.DMA,
                   pltpu.SemaphoreType.DMA]
    scratch = [
        pltpu.VMEM((NCHUNK, B), jnp.int32),
        pltpu.VMEM((NCHUNK, B), jnp.int32),
        *buf(), *buf(), *buf(),
        pltpu.VMEM_SHARED((N, UW), _F32),
    ]

    def body(srcA, dstA, uA, dA, srcB, dstB, uB, dB, zo,
             poA, poB,
             ixs, ixd, ur0, dr0, ms0, g0, s0, ur1, dr1, ms1, g1, s1,
             ur2, dr2, ms2, g2, s2, acc):
        cid = lax.axis_index("c")
        sid = lax.axis_index("s")
        wid = sid * 2 + cid
        rbase = wid * NCHUNK
        iota = lax.iota(jnp.int32, 16)
        colpat = [lax.shift_right_logical(iota, 3) + 2 * v for v in range(4)]
        bufs = ((ur0, dr0, ms0, g0, s0), (ur1, dr1, ms1, g1, s1),
                (ur2, dr2, ms2, g2, s2))

        for (srcR, dstR, uR, dR, poR) in (
                (srcA, dstA, uA, dA, poA),
                (srcB, dstB, uB, dB, poB)):

            @pl.when(sid == 0)
            def _():
                pltpu.sync_copy(zo, acc)
            pltpu.sync_copy(srcR.at[pl.ds(rbase, NCHUNK)], ixs)
            pltpu.sync_copy(dstR.at[pl.ds(rbase, NCHUNK)], ixd)
            plsc.subcore_barrier()

            def fire(i, bf):
                ur, dr, ms, sem, ssem = bf
                pltpu.async_copy(uR.at[ixs.at[i]], ur, sem)
                pltpu.async_copy(dR.at[ixd.at[i]], dr, sem)

            def waitg(i, bf):
                ur, dr, ms, sem, ssem = bf
                pltpu.make_async_copy(uR.at[ixs.at[i]], ur, sem).wait()
                pltpu.make_async_copy(dR.at[ixd.at[i]], dr, sem).wait()

            def drain_scatter(i, bf):
                ur, dr, ms, sem, ssem = bf
                pltpu.make_async_copy(ms, acc.at[ixd.at[i]], ssem).wait()

            def compute(i, bf, first):
                ur, dr, ms, sem, ssem = bf
                if not first:
                    drain_scatter(i, bf)

                @plsc.parallel_loop(0, B, unroll=8)
                def _(b):
                    if wide:
                        sv = ur[b, pl.ds(MID, HP)]
                        dv = dr[b]
                    else:
                        sv = plsc.load_gather(
                            ur, [jnp.full((16,), b, jnp.int32),
                                 jnp.full((16,), MID, jnp.int32)])
                        dv = plsc.load_gather(
                            dr, [jnp.full((16,), b, jnp.int32)])
                    ev = sv + dv
                    ev = jnp.where(ev >= 0.0, ev, NEG * ev)
                    exv = jnp.exp(ev)
                    ms[b, pl.ds(MID, HP)] = exv
                    for v in range(4):
                        sl = pl.ds(v * 16, 16)
                        if wide:
                            ms[b, sl] = ur[b, sl] * exv[colpat[v]]
                        else:
                            ms[b, sl] = ur[b, sl] * exv

                pltpu.async_copy(ms, acc.at[ixd.at[i]], ssem, add=True)

            def fire_guarded(i, bf):
                @pl.when(i < NCHUNK)
                def _():
                    fire(i, bf)

            for j in range(3):
                fire(j, bufs[j])
            for j in range(3):
                waitg(j, bufs[j])
                compute(j, bufs[j], True)
                fire(j + 3, bufs[j])

            def step(k, c):
                i = 3 * k
                for j in range(3):
                    waitg(i + j, bufs[j])
                    compute(i + j, bufs[j], False)
                    fire_guarded(i + j + 3, bufs[j])
                return c

            lax.fori_loop(1, (NCHUNK - 5) // 3 + 1, step, 0)
            waitg(NCHUNK - 2, bufs[0])
            compute(NCHUNK - 2, bufs[0], False)
            waitg(NCHUNK - 1, bufs[1])
            compute(NCHUNK - 1, bufs[1], False)
            drain_scatter(NCHUNK - 3, bufs[2])
            drain_scatter(NCHUNK - 2, bufs[0])
            drain_scatter(NCHUNK - 1, bufs[1])
            plsc.subcore_barrier()

            @pl.when(sid == 0)
            def _():
                pltpu.sync_copy(acc, poR.at[cid])
            plsc.subcore_barrier()

    return pl.kernel(
        body, out_type=out_type, mesh=mesh, scratch_types=scratch,
        compiler_params=pltpu.CompilerParams(
            needs_layout_passes=False, use_tc_tiling_on_sc=False))


_edge8 = _edge_pair(8)
_edge1 = _edge_pair(1)


def _tc2(poA, poB, b1, b2, wg1a, wg1b, expd, w12, as12r, sd12w, w22, as22r,
         sd22w):
    def body(poA_ref, poB_ref, b1_ref, b2_ref, wg1a_ref,
             wg1b_ref, expd_ref, w12_ref, as12r_ref, sd12w_ref, w22_ref,
             as22r_ref, sd22w_ref, u12_ref, d12_ref, u22_ref, d22_ref):
        expd_m = expd_ref[...]
        pA = poA_ref[0] + poA_ref[1]
        o1 = pA[:, :MID]
        rec1 = 1.0 / (pA[:, MID:] + 1e-16)
        ns11 = o1 * jnp.dot(rec1, expd_m, preferred_element_type=_F32) + b1_ref[...]
        ns11 = jnp.where(ns11 > 0.0, ns11, jnp.exp(ns11) - 1.0)
        pB = poB_ref[0] + poB_ref[1]
        o2 = pB[:, :MID]
        rec2 = 1.0 / (pB[:, MID:] + 1e-16)
        ns21 = o2 * jnp.dot(rec2, expd_m, preferred_element_type=_F32) + b2_ref[...]
        ns21 = jnp.where(ns21 > 0.0, ns21, jnp.exp(ns21) - 1.0)
        zl = (jnp.dot(ns11, wg1a_ref[...], preferred_element_type=_F32) +
              jnp.dot(ns21, wg1b_ref[...], preferred_element_type=_F32))
        z = 1.0 / (1.0 + jnp.exp(-zl))
        midv = z * ns11 + (1.0 - z) * ns21
        h12 = jnp.dot(midv, w12_ref[...], preferred_element_type=_F32)
        s12 = jnp.dot(h12, as12r_ref[...], preferred_element_type=_F32)
        u12_ref[...] = jnp.concatenate([h12, s12], axis=1)
        d12_ref[...] = jnp.dot(h12, sd12w_ref[...], preferred_element_type=_F32)
        h22 = jnp.dot(midv, w22_ref[...], preferred_element_type=_F32)
        s22 = jnp.dot(h22, as22r_ref[...], preferred_element_type=_F32)
        u22_ref[...] = jnp.concatenate([h22, s22], axis=1)
        d22_ref[...] = jnp.dot(h22, sd22w_ref[...], preferred_element_type=_F32)

    nb = N // RB
    whole = lambda shape: pl.BlockSpec(shape, lambda i: (0,) * len(shape))
    rows = lambda w: pl.BlockSpec((RB, w), lambda i: (i, 0))
    prow = lambda w: pl.BlockSpec((2, RB, w), lambda i: (0, i, 0))
    return pl.pallas_call(
        body,
        grid=(nb,),
        in_specs=[prow(UW), prow(UW),
                  whole((1, MID)), whole((1, MID)),
                  whole((MID, MID)), whole((MID, MID)), whole((HP, MID)),
                  whole((MID, MID)), whole((MID, HP)), whole((MID, 8)),
                  whole((MID, MID)), whole((MID, HP)), whole((MID, 8))],
        out_specs=[rows(UW), rows(8), rows(UW), rows(8)],
        out_shape=[jax.ShapeDtypeStruct((N, UW), _F32),
                   jax.ShapeDtypeStruct((N, 8), _F32),
                   jax.ShapeDtypeStruct((N, UW), _F32),
                   jax.ShapeDtypeStruct((N, 8), _F32)],
    )(poA, poB, b1, b2, wg1a, wg1b, expd, w12, as12r, sd12w, w22, as22r,
      sd22w)


def _tc3(poA, poB, b1, b2, wg2a, wg2b):
    def body(poA_ref, poB_ref, b1_ref, b2_ref, wg2a_ref, wg2b_ref, out_ref):
        pA = poA_ref[0] + poA_ref[1]
        ns12 = pA[:, :MID] * (1.0 / (pA[:, MID:MID + 1] + 1e-16)) + b1_ref[...]
        pB = poB_ref[0] + poB_ref[1]
        ns22 = pB[:, :MID] * (1.0 / (pB[:, MID:MID + 1] + 1e-16)) + b2_ref[...]
        zl = (jnp.dot(ns12, wg2a_ref[...], preferred_element_type=_F32) +
              jnp.dot(ns22, wg2b_ref[...], preferred_element_type=_F32))
        z = 1.0 / (1.0 + jnp.exp(-zl))
        outv = z * ns12 + (1.0 - z) * ns22
        m = jnp.max(outv, axis=1, keepdims=True)
        sh = outv - m
        lse = jnp.log(jnp.sum(jnp.exp(sh), axis=1, keepdims=True))
        out_ref[...] = sh - lse

    nb = N // RB
    whole = lambda shape: pl.BlockSpec(shape, lambda i: (0,) * len(shape))
    rows = lambda w: pl.BlockSpec((RB, w), lambda i: (i, 0))
    prow = lambda w: pl.BlockSpec((2, RB, w), lambda i: (0, i, 0))
    return pl.pallas_call(
        body,
        grid=(nb,),
        in_specs=[prow(UW), prow(UW),
                  whole((1, MID)), whole((1, MID)),
                  whole((MID, MID)), whole((MID, MID))],
        out_specs=rows(MID),
        out_shape=jax.ShapeDtypeStruct((N, MID), _F32),
    )(poA, poB, b1, b2, wg2a, wg2b)


def _blockdiag(a):
    h, c = a.shape
    eye = jnp.eye(h, dtype=_F32)
    m = (a[:, :, None] * eye[:, None, :]).reshape(h * c, h)
    return jnp.pad(m, ((0, 0), (0, HP - h)))


def kernel(node_feature, one_adj_list, two_adj_list, W11, a_src11, a_dst11,
           b11, W21, a_src21, a_dst21, b21, Wg1, W12, a_src12, a_dst12, b12,
           W22, a_src22, a_dst22, b22, Wg2):
    srcA = one_adj_list[0].reshape(E // B, B)
    dstA = one_adj_list[1].reshape(E // B, B)
    srcB = two_adj_list[0].reshape(E // B, B)
    dstB = two_adj_list[1].reshape(E // B, B)

    u1, d1, u2, d2 = _tc1(
        node_feature, W11, _blockdiag(a_src11), _blockdiag(a_dst11),
        W21, _blockdiag(a_src21), _blockdiag(a_dst21))

    zo = jnp.zeros((N, UW), _F32)
    poA, poB = _edge8(srcA, dstA, u1, d1, srcB, dstB, u2, d2, zo)

    expd = jnp.pad(jnp.repeat(jnp.eye(8, dtype=_F32), 8, axis=1),
                   ((0, HP - 8), (0, 0)))
    as12r = jnp.tile(a_src12.reshape(MID, 1), (1, HP))
    as22r = jnp.tile(a_src22.reshape(MID, 1), (1, HP))
    sd12w = jnp.pad(a_dst12.reshape(MID, 1), ((0, 0), (0, 7)))
    sd22w = jnp.pad(a_dst22.reshape(MID, 1), ((0, 0), (0, 7)))
    u12, d12, u22, d22 = _tc2(
        poA, poB, b11.reshape(1, MID), b21.reshape(1, MID),
        Wg1[:MID], Wg1[MID:], expd, W12, as12r, sd12w, W22, as22r, sd22w)

    poA2, poB2 = _edge1(srcA, dstA, u12, d12[:, 0],
                        srcB, dstB, u22, d22[:, 0], zo)

    return _tc3(poA2, poB2, b12.reshape(1, MID), b22.reshape(1, MID),
                Wg2[:MID], Wg2[MID:])
